# Initial kernel scaffold; baseline (speedup 1.0000x reference)
import jax
import jax.numpy as jnp
from jax.experimental import pallas as pl


def kernel(x, edge_index, edge_weight, W_enc0, b_enc0, W_enc1, b_enc1, W_core0, b_core0, W_core1, b_core1, W_dec0, b_dec0, W_dec1, b_dec1):
    raise NotImplementedError("write your pallas kernel here")


# speedup vs baseline: 4.6621x; 4.6621x over previous
import functools

import jax
import jax.numpy as jnp
from jax import lax
from jax.experimental import pallas as pl
from jax.experimental.pallas import tpu as pltpu
from jax.experimental.pallas import tpu_sc as plsc

_NC, _NS = 2, 16
_NW = _NC * _NS
_CHUNK = 128
_LANES = 16


def _make_agg_kernel(n, f, e):
    epw = e // _NW
    full = epw // _CHUNK
    rem = epw - full * _CHUNK
    io_tiles = _NS
    while io_tiles > 1 and (n % io_tiles or (n // io_tiles) % 8):
        io_tiles -= 1
    rows_per_tile = n // io_tiles
    rchunk = 256 - 256 % 8
    while rows_per_tile % rchunk or rchunk % 8:
        rchunk -= 8
    nrc = rows_per_tile // rchunk
    nsub = f // _LANES

    mesh = plsc.VectorSubcoreMesh(core_axis_name="c", subcore_axis_name="s")

    scratch = [
        pltpu.VMEM_SHARED((n, f), jnp.float32),
        pltpu.VMEM((_CHUNK,), jnp.int32),
        pltpu.VMEM((_CHUNK,), jnp.int32),
        pltpu.VMEM((_CHUNK,), jnp.float32),
        pltpu.VMEM((_CHUNK, f), jnp.float32),
        pltpu.VMEM((rchunk, f), jnp.float32),
        pltpu.SemaphoreType.DMA,
    ]
    if rem:
        scratch += [
            pltpu.VMEM((rem,), jnp.int32),
            pltpu.VMEM((rem,), jnp.int32),
            pltpu.VMEM((rem,), jnp.float32),
            pltpu.VMEM((rem, f), jnp.float32),
        ]

    @functools.partial(
        pl.kernel,
        mesh=mesh,
        out_type=jax.ShapeDtypeStruct((2 * n, f), jnp.float32),
        scratch_types=scratch,
    )
    def agg(h_hbm, src_hbm, dst_hbm, ew_hbm, out_hbm, acc, src_v, dst_v,
            ew_v, rows_v, stage_v, sem, *rem_bufs):
        c = lax.axis_index("c")
        s = lax.axis_index("s")
        wid = c * _NS + s
        row0 = s * rows_per_tile

        @pl.when(s < io_tiles)
        def _zero():
            def zero_body(j, carry):
                for k in range(nsub):
                    stage_v[j, pl.ds(k * _LANES, _LANES)] = jnp.zeros(
                        (_LANES,), jnp.float32)
                return carry
            lax.fori_loop(0, rchunk, zero_body, 0)
            for i in range(nrc):
                pltpu.sync_copy(stage_v,
                                acc.at[pl.ds(row0 + i * rchunk, rchunk)])
        plsc.subcore_barrier()

        ebase = wid * epw

        def chunk_body(i, carry):
            base = ebase + i * _CHUNK
            pltpu.sync_copy(src_hbm.at[pl.ds(base, _CHUNK)], src_v)
            pltpu.sync_copy(dst_hbm.at[pl.ds(base, _CHUNK)], dst_v)
            pltpu.sync_copy(ew_hbm.at[pl.ds(base, _CHUNK)], ew_v)
            pltpu.async_copy(h_hbm.at[src_v], rows_v, sem).wait()

            def scale_body(g, inner):
                wvec = ew_v[pl.ds(g * _LANES, _LANES)]
                for l in range(_LANES):
                    w = jnp.full((_LANES,), wvec[l], jnp.float32)
                    row = g * _LANES + l
                    for k in range(nsub):
                        sl = pl.ds(k * _LANES, _LANES)
                        rows_v[row, sl] = rows_v[row, sl] * w
                return inner
            lax.fori_loop(0, _CHUNK // _LANES, scale_body, 0)
            pltpu.sync_copy(rows_v, acc.at[dst_v], add=True)
            return carry
        lax.fori_loop(0, full, chunk_body, 0)

        if rem:
            srcr_v, dstr_v, ewr_v, rowsr_v = rem_bufs
            base = ebase + full * _CHUNK
            pltpu.sync_copy(src_hbm.at[pl.ds(base, rem)], srcr_v)
            pltpu.sync_copy(dst_hbm.at[pl.ds(base, rem)], dstr_v)
            pltpu.sync_copy(ew_hbm.at[pl.ds(base, rem)], ewr_v)
            pltpu.async_copy(h_hbm.at[srcr_v], rowsr_v, sem).wait()
            for g in range(rem // _LANES):
                wvec = ewr_v[pl.ds(g * _LANES, _LANES)]
                for l in range(_LANES):
                    w = jnp.full((_LANES,), wvec[l], jnp.float32)
                    row = g * _LANES + l
                    for k in range(nsub):
                        sl = pl.ds(k * _LANES, _LANES)
                        rowsr_v[row, sl] = rowsr_v[row, sl] * w
            pltpu.sync_copy(rowsr_v, acc.at[dstr_v], add=True)

        plsc.subcore_barrier()

        @pl.when(s < io_tiles)
        def _writeout():
            out0 = c * n
            for i in range(nrc):
                r = row0 + i * rchunk
                pltpu.sync_copy(acc.at[pl.ds(r, rchunk)], stage_v)
                pltpu.sync_copy(stage_v, out_hbm.at[pl.ds(out0 + r, rchunk)])

    return agg


_BLK = 2000


def _enc_body(x_ref, w0_ref, b0_ref, w1_ref, b1_ref, o_ref):
    t = jnp.dot(x_ref[...], w0_ref[...],
                preferred_element_type=jnp.float32) + b0_ref[...]
    o_ref[...] = jnp.dot(t, w1_ref[...],
                         preferred_element_type=jnp.float32) + b1_ref[...]


def _encoder(x, w0, b0, w1, b1):
    n, fin = x.shape
    l0, l1 = w0.shape[1], w1.shape[1]
    return pl.pallas_call(
        _enc_body,
        grid=(n // _BLK,),
        in_specs=[
            pl.BlockSpec((_BLK, fin), lambda i: (i, 0)),
            pl.BlockSpec((fin, l0), lambda i: (0, 0)),
            pl.BlockSpec((1, l0), lambda i: (0, 0)),
            pl.BlockSpec((l0, l1), lambda i: (0, 0)),
            pl.BlockSpec((1, l1), lambda i: (0, 0)),
        ],
        out_specs=pl.BlockSpec((_BLK, l1), lambda i: (i, 0)),
        out_shape=jax.ShapeDtypeStruct((n, l1), jnp.float32),
    )(x, w0, b0.reshape(1, -1), w1, b1.reshape(1, -1))


def _core_body(a0_ref, a1_ref, h_ref, w_ref, b_ref, o_ref):
    agg = a0_ref[...] + a1_ref[...]
    o_ref[...] = (jnp.dot(agg, w_ref[...],
                          preferred_element_type=jnp.float32)
                  + b_ref[...] + h_ref[...])


def _core_update(agg2, h, w, b):
    n, f = h.shape
    nb = n // _BLK
    return pl.pallas_call(
        _core_body,
        grid=(nb,),
        in_specs=[
            pl.BlockSpec((_BLK, f), lambda i: (i, 0)),
            pl.BlockSpec((_BLK, f), lambda i: (i + nb, 0)),
            pl.BlockSpec((_BLK, f), lambda i: (i, 0)),
            pl.BlockSpec((f, f), lambda i: (0, 0)),
            pl.BlockSpec((1, f), lambda i: (0, 0)),
        ],
        out_specs=pl.BlockSpec((_BLK, f), lambda i: (i, 0)),
        out_shape=jax.ShapeDtypeStruct((n, f), jnp.float32),
    )(agg2, agg2, h, w, b.reshape(1, -1))


def _final_body(a0_ref, a1_ref, h_ref, wc_ref, bc_ref, wd0_ref, bd0_ref,
                wd1_ref, bd1_ref, o_ref):
    hh = (jnp.dot(a0_ref[...] + a1_ref[...], wc_ref[...],
                  preferred_element_type=jnp.float32)
          + bc_ref[...] + h_ref[...])
    hh = jnp.dot(hh, wd0_ref[...],
                 preferred_element_type=jnp.float32) + bd0_ref[...]
    o_ref[...] = jnp.dot(hh, wd1_ref[...],
                         preferred_element_type=jnp.float32) + bd1_ref[...]


def _final(agg2, h, wc, bc, wd0, bd0, wd1p, bd1p):
    n, f = h.shape
    nb = n // _BLK
    return pl.pallas_call(
        _final_body,
        grid=(nb,),
        in_specs=[
            pl.BlockSpec((_BLK, f), lambda i: (i, 0)),
            pl.BlockSpec((_BLK, f), lambda i: (i + nb, 0)),
            pl.BlockSpec((_BLK, f), lambda i: (i, 0)),
            pl.BlockSpec((f, f), lambda i: (0, 0)),
            pl.BlockSpec((1, f), lambda i: (0, 0)),
            pl.BlockSpec((f, f), lambda i: (0, 0)),
            pl.BlockSpec((1, f), lambda i: (0, 0)),
            pl.BlockSpec((f, f), lambda i: (0, 0)),
            pl.BlockSpec((1, f), lambda i: (0, 0)),
        ],
        out_specs=pl.BlockSpec((_BLK, f), lambda i: (i, 0)),
        out_shape=jax.ShapeDtypeStruct((n, f), jnp.float32),
    )(agg2, agg2, h, wc, bc.reshape(1, -1), wd0, bd0.reshape(1, -1),
      wd1p, bd1p.reshape(1, -1))


def kernel(x, edge_index, edge_weight, W_enc0, b_enc0, W_enc1, b_enc1,
           W_core0, b_core0, W_core1, b_core1, W_dec0, b_dec0, W_dec1,
           b_dec1):
    n, f = x.shape
    e = edge_weight.shape[0]
    src = edge_index[1]
    dst = edge_index[0]

    h = _encoder(x, W_enc0, b_enc0, W_enc1, b_enc1)

    agg_fn = _make_agg_kernel(n, f, e)
    agg2 = agg_fn(h, src, dst, edge_weight)
    h = _core_update(agg2, h, W_core0, b_core0)
    agg2 = agg_fn(h, src, dst, edge_weight)

    c = W_dec1.shape[1]
    wd1p = jnp.pad(W_dec1, ((0, 0), (0, f - c)))
    bd1p = jnp.pad(b_dec1, (0, f - c))
    out = _final(agg2, h, W_core1, b_core1, W_dec0, b_dec0, wd1p, bd1p)
    return out[:, :c]


# speedup vs baseline: 6.4451x; 1.3825x over previous
import functools

import jax
import jax.numpy as jnp
from jax import lax
from jax.experimental import pallas as pl
from jax.experimental.pallas import tpu as pltpu
from jax.experimental.pallas import tpu_sc as plsc

_NC, _NS = 2, 16
_NW = _NC * _NS
_CHUNK = 64
_LANES = 16
_RING = 4


def _make_agg_kernel(n, f, e):
    epw = e // _NW
    full = epw // _CHUNK
    rem = epw - full * _CHUNK
    io_tiles = _NS
    while io_tiles > 1 and (n % io_tiles or (n // io_tiles) % 8):
        io_tiles -= 1
    rows_per_tile = n // io_tiles
    rchunk = 40
    while rows_per_tile % rchunk or rchunk % 8:
        rchunk -= 8
    nrc = rows_per_tile // rchunk
    nsub = f // _LANES

    ngrp = full // _RING
    assert ngrp * _RING == full

    mesh = plsc.VectorSubcoreMesh(core_axis_name="c", subcore_axis_name="s")

    scratch = [
        pltpu.VMEM_SHARED((n, f), jnp.float32),
        pltpu.VMEM((rchunk, f), jnp.float32),
    ]
    scratch += [pltpu.VMEM((_CHUNK, f), jnp.float32) for _ in range(_RING)]
    scratch += [pltpu.VMEM((_CHUNK,), jnp.int32) for _ in range(_RING)]
    scratch += [pltpu.VMEM((_CHUNK,), jnp.int32) for _ in range(_RING)]
    scratch += [pltpu.VMEM((_CHUNK,), jnp.float32) for _ in range(_RING)]
    scratch += [
        pltpu.SemaphoreType.DMA((_RING,)),
        pltpu.SemaphoreType.DMA((_RING,)),
        pltpu.SemaphoreType.DMA((_RING,)),
    ]
    if rem:
        scratch += [
            pltpu.VMEM((rem,), jnp.int32),
            pltpu.VMEM((rem,), jnp.int32),
            pltpu.VMEM((rem,), jnp.float32),
            pltpu.VMEM((rem, f), jnp.float32),
        ]

    @functools.partial(
        pl.kernel,
        mesh=mesh,
        out_type=jax.ShapeDtypeStruct((2 * n, f), jnp.float32),
        scratch_types=scratch,
    )
    def agg(h_hbm, src_hbm, dst_hbm, ew_hbm, out_hbm, acc, stage_v, *bufs):
        rows_r = bufs[:_RING]
        src_r = bufs[_RING:2 * _RING]
        dst_r = bufs[2 * _RING:3 * _RING]
        ew_r = bufs[3 * _RING:4 * _RING]
        gsem, isem, ssem = bufs[4 * _RING:4 * _RING + 3]
        rem_bufs = bufs[4 * _RING + 3:]
        c = lax.axis_index("c")
        s = lax.axis_index("s")
        wid = c * _NS + s
        row0 = s * rows_per_tile
        ebase = wid * epw

        def idx_descs(ci, b):
            base = ebase + ci * _CHUNK
            return (
                pltpu.make_async_copy(src_hbm.at[pl.ds(base, _CHUNK)],
                                      src_r[b], isem.at[b]),
                pltpu.make_async_copy(dst_hbm.at[pl.ds(base, _CHUNK)],
                                      dst_r[b], isem.at[b]),
                pltpu.make_async_copy(ew_hbm.at[pl.ds(base, _CHUNK)],
                                      ew_r[b], isem.at[b]),
            )

        def issue_idx(ci, b):
            for d in idx_descs(ci, b):
                d.start()

        def wait_idx(ci, b):
            for d in idx_descs(ci, b):
                d.wait()

        def gather_desc(b):
            return pltpu.make_async_copy(h_hbm.at[src_r[b]], rows_r[b],
                                         gsem.at[b])

        def scatter_desc(b):
            return pltpu.make_async_copy(rows_r[b], acc.at[dst_r[b]],
                                         ssem.at[b])

        def issue_scatter(b):
            pltpu.async_copy(rows_r[b], acc.at[dst_r[b]], ssem.at[b],
                             add=True)

        def scale(b):
            def scale_body(g, inner):
                wvec = ew_r[b][pl.ds(g * _LANES, _LANES)]
                for l in range(_LANES):
                    w = jnp.full((_LANES,), wvec[l], jnp.float32)
                    row = g * _LANES + l
                    for k in range(nsub):
                        sl = pl.ds(k * _LANES, _LANES)
                        rows_r[b][row, sl] = rows_r[b][row, sl] * w
                return inner
            lax.fori_loop(0, _CHUNK // _LANES, scale_body, 0)

        issue_idx(0, 0)
        issue_idx(1, 1)

        @pl.when(s < io_tiles)
        def _zero():
            def zero_body(j, carry):
                for k in range(nsub):
                    stage_v[j, pl.ds(k * _LANES, _LANES)] = jnp.zeros(
                        (_LANES,), jnp.float32)
                return carry
            lax.fori_loop(0, rchunk, zero_body, 0)
            for i in range(nrc):
                pltpu.sync_copy(stage_v,
                                acc.at[pl.ds(row0 + i * rchunk, rchunk)])
        plsc.subcore_barrier()

        wait_idx(0, 0)
        gather_desc(0).start()

        def group_body(g9, carry):
            for j in range(_RING):
                b = j
                ci = g9 * _RING + j
                gather_desc(b).wait()
                scale(b)
                sb = (j + 2) % _RING
                if j >= 2:
                    scatter_desc(sb).wait()
                else:
                    @pl.when(g9 > 0)
                    def _ws():
                        scatter_desc(sb).wait()
                gb = (j + 1) % _RING
                if j < _RING - 1:
                    wait_idx(ci + 1, gb)
                    gather_desc(gb).start()
                else:
                    @pl.when(g9 < ngrp - 1)
                    def _wg():
                        wait_idx(ci + 1, gb)
                        gather_desc(gb).start()
                if j < _RING - 2:
                    issue_idx(ci + 2, sb)
                else:
                    @pl.when(g9 < ngrp - 1)
                    def _wi():
                        issue_idx(ci + 2, sb)
                issue_scatter(b)
            return carry
        lax.fori_loop(0, ngrp, group_body, 0)
        scatter_desc(_RING - 2).wait()
        scatter_desc(_RING - 1).wait()

        if rem:
            srcr_v, dstr_v, ewr_v, rowsr_v = rem_bufs
            base = ebase + full * _CHUNK
            pltpu.sync_copy(src_hbm.at[pl.ds(base, rem)], srcr_v)
            pltpu.sync_copy(dst_hbm.at[pl.ds(base, rem)], dstr_v)
            pltpu.sync_copy(ew_hbm.at[pl.ds(base, rem)], ewr_v)
            pltpu.async_copy(h_hbm.at[srcr_v], rowsr_v, gsem.at[0]).wait()
            for g in range(rem // _LANES):
                wvec = ewr_v[pl.ds(g * _LANES, _LANES)]
                for l in range(_LANES):
                    w = jnp.full((_LANES,), wvec[l], jnp.float32)
                    row = g * _LANES + l
                    for k in range(nsub):
                        sl = pl.ds(k * _LANES, _LANES)
                        rowsr_v[row, sl] = rowsr_v[row, sl] * w
            pltpu.sync_copy(rowsr_v, acc.at[dstr_v], add=True)

        plsc.subcore_barrier()

        @pl.when(s < io_tiles)
        def _writeout():
            out0 = c * n
            for i in range(nrc):
                r = row0 + i * rchunk
                pltpu.sync_copy(acc.at[pl.ds(r, rchunk)], stage_v)
                pltpu.sync_copy(stage_v, out_hbm.at[pl.ds(out0 + r, rchunk)])

    return agg


_BLK = 2000


def _enc_body(x_ref, w0_ref, b0_ref, w1_ref, b1_ref, o_ref):
    t = jnp.dot(x_ref[...], w0_ref[...],
                preferred_element_type=jnp.float32) + b0_ref[...]
    o_ref[...] = jnp.dot(t, w1_ref[...],
                         preferred_element_type=jnp.float32) + b1_ref[...]


def _encoder(x, w0, b0, w1, b1):
    n, fin = x.shape
    l0, l1 = w0.shape[1], w1.shape[1]
    return pl.pallas_call(
        _enc_body,
        grid=(n // _BLK,),
        in_specs=[
            pl.BlockSpec((_BLK, fin), lambda i: (i, 0)),
            pl.BlockSpec((fin, l0), lambda i: (0, 0)),
            pl.BlockSpec((1, l0), lambda i: (0, 0)),
            pl.BlockSpec((l0, l1), lambda i: (0, 0)),
            pl.BlockSpec((1, l1), lambda i: (0, 0)),
        ],
        out_specs=pl.BlockSpec((_BLK, l1), lambda i: (i, 0)),
        out_shape=jax.ShapeDtypeStruct((n, l1), jnp.float32),
    )(x, w0, b0.reshape(1, -1), w1, b1.reshape(1, -1))


def _core_body(a0_ref, a1_ref, h_ref, w_ref, b_ref, o_ref):
    agg = a0_ref[...] + a1_ref[...]
    o_ref[...] = (jnp.dot(agg, w_ref[...],
                          preferred_element_type=jnp.float32)
                  + b_ref[...] + h_ref[...])


def _core_update(agg2, h, w, b):
    n, f = h.shape
    nb = n // _BLK
    return pl.pallas_call(
        _core_body,
        grid=(nb,),
        in_specs=[
            pl.BlockSpec((_BLK, f), lambda i: (i, 0)),
            pl.BlockSpec((_BLK, f), lambda i: (i + nb, 0)),
            pl.BlockSpec((_BLK, f), lambda i: (i, 0)),
            pl.BlockSpec((f, f), lambda i: (0, 0)),
            pl.BlockSpec((1, f), lambda i: (0, 0)),
        ],
        out_specs=pl.BlockSpec((_BLK, f), lambda i: (i, 0)),
        out_shape=jax.ShapeDtypeStruct((n, f), jnp.float32),
    )(agg2, agg2, h, w, b.reshape(1, -1))


def _final_body(a0_ref, a1_ref, h_ref, wc_ref, bc_ref, wd0_ref, bd0_ref,
                wd1_ref, bd1_ref, o_ref):
    hh = (jnp.dot(a0_ref[...] + a1_ref[...], wc_ref[...],
                  preferred_element_type=jnp.float32)
          + bc_ref[...] + h_ref[...])
    hh = jnp.dot(hh, wd0_ref[...],
                 preferred_element_type=jnp.float32) + bd0_ref[...]
    o_ref[...] = jnp.dot(hh, wd1_ref[...],
                         preferred_element_type=jnp.float32) + bd1_ref[...]


def _final(agg2, h, wc, bc, wd0, bd0, wd1p, bd1p):
    n, f = h.shape
    nb = n // _BLK
    return pl.pallas_call(
        _final_body,
        grid=(nb,),
        in_specs=[
            pl.BlockSpec((_BLK, f), lambda i: (i, 0)),
            pl.BlockSpec((_BLK, f), lambda i: (i + nb, 0)),
            pl.BlockSpec((_BLK, f), lambda i: (i, 0)),
            pl.BlockSpec((f, f), lambda i: (0, 0)),
            pl.BlockSpec((1, f), lambda i: (0, 0)),
            pl.BlockSpec((f, f), lambda i: (0, 0)),
            pl.BlockSpec((1, f), lambda i: (0, 0)),
            pl.BlockSpec((f, f), lambda i: (0, 0)),
            pl.BlockSpec((1, f), lambda i: (0, 0)),
        ],
        out_specs=pl.BlockSpec((_BLK, f), lambda i: (i, 0)),
        out_shape=jax.ShapeDtypeStruct((n, f), jnp.float32),
    )(agg2, agg2, h, wc, bc.reshape(1, -1), wd0, bd0.reshape(1, -1),
      wd1p, bd1p.reshape(1, -1))


def kernel(x, edge_index, edge_weight, W_enc0, b_enc0, W_enc1, b_enc1,
           W_core0, b_core0, W_core1, b_core1, W_dec0, b_dec0, W_dec1,
           b_dec1):
    n, f = x.shape
    e = edge_weight.shape[0]
    src = edge_index[1]
    dst = edge_index[0]

    h = _encoder(x, W_enc0, b_enc0, W_enc1, b_enc1)

    agg_fn = _make_agg_kernel(n, f, e)
    agg2 = agg_fn(h, src, dst, edge_weight)
    h = _core_update(agg2, h, W_core0, b_core0)
    agg2 = agg_fn(h, src, dst, edge_weight)

    c = W_dec1.shape[1]
    wd1p = jnp.pad(W_dec1, ((0, 0), (0, f - c)))
    bd1p = jnp.pad(b_dec1, (0, f - c))
    out = _final(agg2, h, W_core1, b_core1, W_dec0, b_dec0, wd1p, bd1p)
    return out[:, :c]


# speedup vs baseline: 10.0348x; 1.5570x over previous
import functools

import jax
import jax.numpy as jnp
from jax import lax
from jax.experimental import pallas as pl
from jax.experimental.pallas import tpu as pltpu
from jax.experimental.pallas import tpu_sc as plsc

_NC, _NS = 2, 16
_NW = _NC * _NS
_CHUNK = 64
_LANES = 16
_RING = 4


def _make_agg_kernel(n, f, e):
    epw = e // _NW
    full = epw // _CHUNK
    rem = epw - full * _CHUNK
    io_tiles = _NS
    while io_tiles > 1 and (n % io_tiles or (n // io_tiles) % 8):
        io_tiles -= 1
    rows_per_tile = n // io_tiles
    rchunk = 40
    while rows_per_tile % rchunk or rchunk % 8:
        rchunk -= 8
    nrc = rows_per_tile // rchunk
    nsub = f // _LANES

    ngrp = full // _RING
    assert ngrp * _RING == full

    mesh = plsc.VectorSubcoreMesh(core_axis_name="c", subcore_axis_name="s")

    scratch = [
        pltpu.VMEM_SHARED((n, f), jnp.float32),
        pltpu.VMEM((rchunk, f), jnp.float32),
    ]
    scratch += [pltpu.VMEM((_CHUNK, f), jnp.float32) for _ in range(_RING)]
    scratch += [pltpu.VMEM((_CHUNK,), jnp.int32) for _ in range(_RING)]
    scratch += [pltpu.VMEM((_CHUNK,), jnp.int32) for _ in range(_RING)]
    scratch += [pltpu.VMEM((_CHUNK,), jnp.float32) for _ in range(_RING)]
    scratch += [
        pltpu.SemaphoreType.DMA((_RING,)),
        pltpu.SemaphoreType.DMA((_RING,)),
        pltpu.SemaphoreType.DMA((_RING,)),
        pltpu.SemaphoreType.DMA((_RING,)),
    ]
    if rem:
        scratch += [
            pltpu.VMEM((rem,), jnp.int32),
            pltpu.VMEM((rem,), jnp.int32),
            pltpu.VMEM((rem,), jnp.float32),
            pltpu.VMEM((rem, f), jnp.float32),
        ]

    @functools.partial(
        pl.kernel,
        mesh=mesh,
        out_type=jax.ShapeDtypeStruct((2 * n, f), jnp.float32),
        scratch_types=scratch,
    )
    def agg(h_hbm, src_hbm, dst_hbm, ew_hbm, out_hbm, acc, stage_v, *bufs):
        rows_r = bufs[:_RING]
        src_r = bufs[_RING:2 * _RING]
        dst_r = bufs[2 * _RING:3 * _RING]
        ew_r = bufs[3 * _RING:4 * _RING]
        gsem, isem, dsem, ssem = bufs[4 * _RING:4 * _RING + 4]
        rem_bufs = bufs[4 * _RING + 4:]
        c = lax.axis_index("c")
        s = lax.axis_index("s")
        wid = c * _NS + s
        row0 = s * rows_per_tile
        ebase = wid * epw

        def srcew_descs(ci, b):
            base = ebase + ci * _CHUNK
            return (
                pltpu.make_async_copy(src_hbm.at[pl.ds(base, _CHUNK)],
                                      src_r[b], isem.at[b]),
                pltpu.make_async_copy(ew_hbm.at[pl.ds(base, _CHUNK)],
                                      ew_r[b], isem.at[b]),
            )

        def dst_desc(ci, b):
            base = ebase + ci * _CHUNK
            return pltpu.make_async_copy(dst_hbm.at[pl.ds(base, _CHUNK)],
                                         dst_r[b], dsem.at[b])

        def issue_srcew(ci, b):
            for d in srcew_descs(ci, b):
                d.start()

        def wait_srcew(ci, b):
            for d in srcew_descs(ci, b):
                d.wait()

        def gather_desc(b):
            return pltpu.make_async_copy(h_hbm.at[src_r[b]], rows_r[b],
                                         gsem.at[b])

        def scatter_desc(b):
            return pltpu.make_async_copy(rows_r[b], acc.at[dst_r[b]],
                                         ssem.at[b])

        def issue_scatter(b):
            pltpu.async_copy(rows_r[b], acc.at[dst_r[b]], ssem.at[b],
                             add=True)

        def scale(b):
            def scale_body(g, inner):
                wvec = ew_r[b][pl.ds(g * _LANES, _LANES)]
                for l in range(_LANES):
                    w = jnp.full((_LANES,), wvec[l], jnp.float32)
                    row = g * _LANES + l
                    for k in range(nsub):
                        sl = pl.ds(k * _LANES, _LANES)
                        rows_r[b][row, sl] = rows_r[b][row, sl] * w
                return inner
            lax.fori_loop(0, _CHUNK // _LANES, scale_body, 0)

        issue_srcew(0, 0)
        issue_srcew(1, 1)
        issue_srcew(2, 2)
        dst_desc(0, 0).start()
        dst_desc(1, 1).start()

        @pl.when(s < io_tiles)
        def _zero():
            def zero_body(j, carry):
                for k in range(nsub):
                    stage_v[j, pl.ds(k * _LANES, _LANES)] = jnp.zeros(
                        (_LANES,), jnp.float32)
                return carry
            lax.fori_loop(0, rchunk, zero_body, 0)
            for i in range(nrc):
                pltpu.sync_copy(stage_v,
                                acc.at[pl.ds(row0 + i * rchunk, rchunk)])
        plsc.subcore_barrier()

        wait_srcew(0, 0)
        gather_desc(0).start()
        wait_srcew(1, 1)
        gather_desc(1).start()

        def group_body(g9, carry):
            for j in range(_RING):
                b = j
                ci = g9 * _RING + j
                gather_desc(b).wait()
                scale(b)
                sb = (j + 2) % _RING
                if j >= 2:
                    scatter_desc(sb).wait()
                else:
                    @pl.when(g9 > 0)
                    def _ws():
                        scatter_desc(sb).wait()
                if j < 2:
                    wait_srcew(ci + 2, sb)
                    gather_desc(sb).start()
                else:
                    @pl.when(g9 < ngrp - 1)
                    def _wg():
                        wait_srcew(ci + 2, sb)
                        gather_desc(sb).start()
                ib = (j + 3) % _RING
                if j < 1:
                    issue_srcew(ci + 3, ib)
                else:
                    @pl.when(g9 < ngrp - 1)
                    def _wi():
                        issue_srcew(ci + 3, ib)
                dst_desc(ci, b).wait()
                issue_scatter(b)
                if j < 2:
                    dst_desc(ci + 2, sb).start()
                else:
                    @pl.when(g9 < ngrp - 1)
                    def _wd():
                        dst_desc(ci + 2, sb).start()
            return carry
        lax.fori_loop(0, ngrp, group_body, 0)
        scatter_desc(_RING - 2).wait()
        scatter_desc(_RING - 1).wait()

        if rem:
            srcr_v, dstr_v, ewr_v, rowsr_v = rem_bufs
            base = ebase + full * _CHUNK
            pltpu.sync_copy(src_hbm.at[pl.ds(base, rem)], srcr_v)
            pltpu.sync_copy(dst_hbm.at[pl.ds(base, rem)], dstr_v)
            pltpu.sync_copy(ew_hbm.at[pl.ds(base, rem)], ewr_v)
            pltpu.async_copy(h_hbm.at[srcr_v], rowsr_v, gsem.at[0]).wait()
            for g in range(rem // _LANES):
                wvec = ewr_v[pl.ds(g * _LANES, _LANES)]
                for l in range(_LANES):
                    w = jnp.full((_LANES,), wvec[l], jnp.float32)
                    row = g * _LANES + l
                    for k in range(nsub):
                        sl = pl.ds(k * _LANES, _LANES)
                        rowsr_v[row, sl] = rowsr_v[row, sl] * w
            pltpu.sync_copy(rowsr_v, acc.at[dstr_v], add=True)

        plsc.subcore_barrier()

        @pl.when(s < io_tiles)
        def _writeout():
            out0 = c * n
            for i in range(nrc):
                r = row0 + i * rchunk
                pltpu.sync_copy(acc.at[pl.ds(r, rchunk)], stage_v)
                pltpu.sync_copy(stage_v, out_hbm.at[pl.ds(out0 + r, rchunk)])

    return agg


_BLK = 2000


def _enc_body(x_ref, w0_ref, b0_ref, w1_ref, b1_ref, o_ref):
    t = jnp.dot(x_ref[...], w0_ref[...],
                preferred_element_type=jnp.float32) + b0_ref[...]
    o_ref[...] = jnp.dot(t, w1_ref[...],
                         preferred_element_type=jnp.float32) + b1_ref[...]


def _encoder(x, w0, b0, w1, b1):
    n, fin = x.shape
    l0, l1 = w0.shape[1], w1.shape[1]
    return pl.pallas_call(
        _enc_body,
        grid=(n // _BLK,),
        in_specs=[
            pl.BlockSpec((_BLK, fin), lambda i: (i, 0)),
            pl.BlockSpec((fin, l0), lambda i: (0, 0)),
            pl.BlockSpec((1, l0), lambda i: (0, 0)),
            pl.BlockSpec((l0, l1), lambda i: (0, 0)),
            pl.BlockSpec((1, l1), lambda i: (0, 0)),
        ],
        out_specs=pl.BlockSpec((_BLK, l1), lambda i: (i, 0)),
        out_shape=jax.ShapeDtypeStruct((n, l1), jnp.float32),
    )(x, w0, b0.reshape(1, -1), w1, b1.reshape(1, -1))


def _core_body(a0_ref, a1_ref, h_ref, w_ref, b_ref, o_ref):
    agg = a0_ref[...] + a1_ref[...]
    o_ref[...] = (jnp.dot(agg, w_ref[...],
                          preferred_element_type=jnp.float32)
                  + b_ref[...] + h_ref[...])


def _core_update(agg2, h, w, b):
    n, f = h.shape
    nb = n // _BLK
    return pl.pallas_call(
        _core_body,
        grid=(nb,),
        in_specs=[
            pl.BlockSpec((_BLK, f), lambda i: (i, 0)),
            pl.BlockSpec((_BLK, f), lambda i: (i + nb, 0)),
            pl.BlockSpec((_BLK, f), lambda i: (i, 0)),
            pl.BlockSpec((f, f), lambda i: (0, 0)),
            pl.BlockSpec((1, f), lambda i: (0, 0)),
        ],
        out_specs=pl.BlockSpec((_BLK, f), lambda i: (i, 0)),
        out_shape=jax.ShapeDtypeStruct((n, f), jnp.float32),
    )(agg2, agg2, h, w, b.reshape(1, -1))


def _final_body(a0_ref, a1_ref, h_ref, wc_ref, bc_ref, wd0_ref, bd0_ref,
                wd1_ref, bd1_ref, o_ref):
    hh = (jnp.dot(a0_ref[...] + a1_ref[...], wc_ref[...],
                  preferred_element_type=jnp.float32)
          + bc_ref[...] + h_ref[...])
    hh = jnp.dot(hh, wd0_ref[...],
                 preferred_element_type=jnp.float32) + bd0_ref[...]
    o_ref[...] = jnp.dot(hh, wd1_ref[...],
                         preferred_element_type=jnp.float32) + bd1_ref[...]


def _final(agg2, h, wc, bc, wd0, bd0, wd1p, bd1p):
    n, f = h.shape
    nb = n // _BLK
    return pl.pallas_call(
        _final_body,
        grid=(nb,),
        in_specs=[
            pl.BlockSpec((_BLK, f), lambda i: (i, 0)),
            pl.BlockSpec((_BLK, f), lambda i: (i + nb, 0)),
            pl.BlockSpec((_BLK, f), lambda i: (i, 0)),
            pl.BlockSpec((f, f), lambda i: (0, 0)),
            pl.BlockSpec((1, f), lambda i: (0, 0)),
            pl.BlockSpec((f, f), lambda i: (0, 0)),
            pl.BlockSpec((1, f), lambda i: (0, 0)),
            pl.BlockSpec((f, f), lambda i: (0, 0)),
            pl.BlockSpec((1, f), lambda i: (0, 0)),
        ],
        out_specs=pl.BlockSpec((_BLK, f), lambda i: (i, 0)),
        out_shape=jax.ShapeDtypeStruct((n, f), jnp.float32),
    )(agg2, agg2, h, wc, bc.reshape(1, -1), wd0, bd0.reshape(1, -1),
      wd1p, bd1p.reshape(1, -1))


def kernel(x, edge_index, edge_weight, W_enc0, b_enc0, W_enc1, b_enc1,
           W_core0, b_core0, W_core1, b_core1, W_dec0, b_dec0, W_dec1,
           b_dec1):
    n, f = x.shape
    e = edge_weight.shape[0]
    src = edge_index[1]
    dst = edge_index[0]

    h = _encoder(x, W_enc0, b_enc0, W_enc1, b_enc1)

    agg_fn = _make_agg_kernel(n, f, e)
    agg2 = agg_fn(h, src, dst, edge_weight)
    h = _core_update(agg2, h, W_core0, b_core0)
    agg2 = agg_fn(h, src, dst, edge_weight)

    c = W_dec1.shape[1]
    wd1p = jnp.pad(W_dec1, ((0, 0), (0, f - c)))
    bd1p = jnp.pad(b_dec1, (0, f - c))
    out = _final(agg2, h, W_core1, b_core1, W_dec0, b_dec0, wd1p, bd1p)
    return out[:, :c]


# speedup vs baseline: 10.4567x; 1.0420x over previous
import functools

import jax
import jax.numpy as jnp
from jax import lax
from jax.experimental import pallas as pl
from jax.experimental.pallas import tpu as pltpu
from jax.experimental.pallas import tpu_sc as plsc

_NC, _NS = 2, 16
_NW = _NC * _NS
_CHUNK = 64
_LANES = 16
_RING = 4


def _make_agg_kernel(n, f, e):
    epw = e // _NW
    full = epw // _CHUNK
    rem = epw - full * _CHUNK
    io_tiles = _NS
    while io_tiles > 1 and (n % io_tiles or (n // io_tiles) % 8):
        io_tiles -= 1
    rows_per_tile = n // io_tiles
    rchunk = 40
    while rows_per_tile % rchunk or rchunk % 8:
        rchunk -= 8
    nrc = rows_per_tile // rchunk
    nsub = f // _LANES

    ngrp = full // _RING
    assert ngrp * _RING == full

    mesh = plsc.VectorSubcoreMesh(core_axis_name="c", subcore_axis_name="s")

    scratch = [
        pltpu.VMEM_SHARED((n, f), jnp.float32),
        pltpu.VMEM((rchunk, f), jnp.float32),
        pltpu.VMEM((rchunk, f), jnp.float32),
        pltpu.SemaphoreType.DMA,
    ]
    scratch += [pltpu.VMEM((_CHUNK, f), jnp.float32) for _ in range(_RING)]
    scratch += [pltpu.VMEM((_CHUNK,), jnp.int32) for _ in range(_RING)]
    scratch += [pltpu.VMEM((_CHUNK,), jnp.int32) for _ in range(_RING)]
    scratch += [pltpu.VMEM((_CHUNK,), jnp.float32) for _ in range(_RING)]
    scratch += [
        pltpu.SemaphoreType.DMA((_RING,)),
        pltpu.SemaphoreType.DMA((_RING,)),
        pltpu.SemaphoreType.DMA((_RING,)),
        pltpu.SemaphoreType.DMA((_RING,)),
    ]
    if rem:
        scratch += [
            pltpu.VMEM((rem,), jnp.int32),
            pltpu.VMEM((rem,), jnp.int32),
            pltpu.VMEM((rem,), jnp.float32),
            pltpu.VMEM((rem, f), jnp.float32),
        ]

    @functools.partial(
        pl.kernel,
        mesh=mesh,
        out_type=jax.ShapeDtypeStruct((2 * n, f), jnp.float32),
        scratch_types=scratch,
    )
    def agg(h_hbm, src_hbm, dst_hbm, ew_hbm, out_hbm, acc, stage_v, stage2_v,
            zsem, *bufs):
        rows_r = bufs[:_RING]
        src_r = bufs[_RING:2 * _RING]
        dst_r = bufs[2 * _RING:3 * _RING]
        ew_r = bufs[3 * _RING:4 * _RING]
        gsem, isem, dsem, ssem = bufs[4 * _RING:4 * _RING + 4]
        rem_bufs = bufs[4 * _RING + 4:]
        c = lax.axis_index("c")
        s = lax.axis_index("s")
        wid = c * _NS + s
        row0 = s * rows_per_tile
        ebase = wid * epw

        def srcew_descs(ci, b):
            base = ebase + ci * _CHUNK
            return (
                pltpu.make_async_copy(src_hbm.at[pl.ds(base, _CHUNK)],
                                      src_r[b], isem.at[b]),
                pltpu.make_async_copy(ew_hbm.at[pl.ds(base, _CHUNK)],
                                      ew_r[b], isem.at[b]),
            )

        def dst_desc(ci, b):
            base = ebase + ci * _CHUNK
            return pltpu.make_async_copy(dst_hbm.at[pl.ds(base, _CHUNK)],
                                         dst_r[b], dsem.at[b])

        def issue_srcew(ci, b):
            for d in srcew_descs(ci, b):
                d.start()

        def wait_srcew(ci, b):
            for d in srcew_descs(ci, b):
                d.wait()

        half = _CHUNK // 2

        def gather_descs(b):
            return (
                pltpu.make_async_copy(
                    h_hbm.at[src_r[b].at[pl.ds(0, half)]],
                    rows_r[b].at[pl.ds(0, half)], gsem.at[b]),
                pltpu.make_async_copy(
                    h_hbm.at[src_r[b].at[pl.ds(half, half)]],
                    rows_r[b].at[pl.ds(half, half)], gsem.at[b]),
            )

        def start_gather(b):
            for d in gather_descs(b):
                d.start()

        def wait_gather(b):
            for d in gather_descs(b):
                d.wait()

        def scatter_desc(b):
            return pltpu.make_async_copy(rows_r[b], acc.at[dst_r[b]],
                                         ssem.at[b])

        def issue_scatter(b):
            pltpu.async_copy(rows_r[b], acc.at[dst_r[b]], ssem.at[b],
                             add=True)

        def scale(b):
            def scale_body(g, inner):
                wvec = ew_r[b][pl.ds(g * _LANES, _LANES)]
                for l in range(_LANES):
                    w = jnp.full((_LANES,), wvec[l], jnp.float32)
                    row = g * _LANES + l
                    for k in range(nsub):
                        sl = pl.ds(k * _LANES, _LANES)
                        rows_r[b][row, sl] = rows_r[b][row, sl] * w
                return inner
            lax.fori_loop(0, _CHUNK // _LANES, scale_body, 0)

        issue_srcew(0, 0)
        issue_srcew(1, 1)
        issue_srcew(2, 2)
        dst_desc(0, 0).start()
        dst_desc(1, 1).start()

        @pl.when(s < io_tiles)
        def _zero():
            def zero_body(j, carry):
                for k in range(nsub):
                    stage_v[j, pl.ds(k * _LANES, _LANES)] = jnp.zeros(
                        (_LANES,), jnp.float32)
                return carry
            lax.fori_loop(0, rchunk, zero_body, 0)
            for i in range(nrc):
                pltpu.async_copy(stage_v,
                                 acc.at[pl.ds(row0 + i * rchunk, rchunk)],
                                 zsem)
            for i in range(nrc):
                pltpu.make_async_copy(
                    stage_v, acc.at[pl.ds(row0 + i * rchunk, rchunk)],
                    zsem).wait()
        plsc.subcore_barrier()

        wait_srcew(0, 0)
        start_gather(0)
        wait_srcew(1, 1)
        start_gather(1)

        def group_body(g9, carry):
            for j in range(_RING):
                b = j
                ci = g9 * _RING + j
                wait_gather(b)
                scale(b)
                sb = (j + 2) % _RING
                if j >= 2:
                    scatter_desc(sb).wait()
                else:
                    @pl.when(g9 > 0)
                    def _ws():
                        scatter_desc(sb).wait()
                if j < 2:
                    wait_srcew(ci + 2, sb)
                    start_gather(sb)
                else:
                    @pl.when(g9 < ngrp - 1)
                    def _wg():
                        wait_srcew(ci + 2, sb)
                        start_gather(sb)
                ib = (j + 3) % _RING
                if j < 1:
                    issue_srcew(ci + 3, ib)
                else:
                    @pl.when(g9 < ngrp - 1)
                    def _wi():
                        issue_srcew(ci + 3, ib)
                dst_desc(ci, b).wait()
                issue_scatter(b)
                if j < 2:
                    dst_desc(ci + 2, sb).start()
                else:
                    @pl.when(g9 < ngrp - 1)
                    def _wd():
                        dst_desc(ci + 2, sb).start()
            return carry
        lax.fori_loop(0, ngrp, group_body, 0)
        scatter_desc(_RING - 2).wait()
        scatter_desc(_RING - 1).wait()

        if rem:
            srcr_v, dstr_v, ewr_v, rowsr_v = rem_bufs
            base = ebase + full * _CHUNK
            pltpu.sync_copy(src_hbm.at[pl.ds(base, rem)], srcr_v)
            pltpu.sync_copy(dst_hbm.at[pl.ds(base, rem)], dstr_v)
            pltpu.sync_copy(ew_hbm.at[pl.ds(base, rem)], ewr_v)
            pltpu.async_copy(h_hbm.at[srcr_v], rowsr_v, gsem.at[0]).wait()
            for g in range(rem // _LANES):
                wvec = ewr_v[pl.ds(g * _LANES, _LANES)]
                for l in range(_LANES):
                    w = jnp.full((_LANES,), wvec[l], jnp.float32)
                    row = g * _LANES + l
                    for k in range(nsub):
                        sl = pl.ds(k * _LANES, _LANES)
                        rowsr_v[row, sl] = rowsr_v[row, sl] * w
            pltpu.sync_copy(rowsr_v, acc.at[dstr_v], add=True)

        plsc.subcore_barrier()

        @pl.when(s < io_tiles)
        def _writeout():
            out0 = c * n
            stages = (stage_v, stage2_v)
            for i in range(nrc):
                r = row0 + i * rchunk
                bb = stages[i % 2]
                if i >= 2:
                    rp = row0 + (i - 2) * rchunk
                    pltpu.make_async_copy(
                        bb, out_hbm.at[pl.ds(out0 + rp, rchunk)], zsem).wait()
                pltpu.sync_copy(acc.at[pl.ds(r, rchunk)], bb)
                pltpu.async_copy(bb, out_hbm.at[pl.ds(out0 + r, rchunk)], zsem)
            for i in range(max(0, nrc - 2), nrc):
                r = row0 + i * rchunk
                pltpu.make_async_copy(
                    stages[i % 2], out_hbm.at[pl.ds(out0 + r, rchunk)],
                    zsem).wait()

    return agg


_BLK = 2000


def _enc_body(x_ref, w0_ref, b0_ref, w1_ref, b1_ref, o_ref):
    t = jnp.dot(x_ref[...], w0_ref[...],
                preferred_element_type=jnp.float32) + b0_ref[...]
    o_ref[...] = jnp.dot(t, w1_ref[...],
                         preferred_element_type=jnp.float32) + b1_ref[...]


def _encoder(x, w0, b0, w1, b1):
    n, fin = x.shape
    l0, l1 = w0.shape[1], w1.shape[1]
    return pl.pallas_call(
        _enc_body,
        grid=(n // _BLK,),
        in_specs=[
            pl.BlockSpec((_BLK, fin), lambda i: (i, 0)),
            pl.BlockSpec((fin, l0), lambda i: (0, 0)),
            pl.BlockSpec((1, l0), lambda i: (0, 0)),
            pl.BlockSpec((l0, l1), lambda i: (0, 0)),
            pl.BlockSpec((1, l1), lambda i: (0, 0)),
        ],
        out_specs=pl.BlockSpec((_BLK, l1), lambda i: (i, 0)),
        out_shape=jax.ShapeDtypeStruct((n, l1), jnp.float32),
    )(x, w0, b0.reshape(1, -1), w1, b1.reshape(1, -1))


def _core_body(a0_ref, a1_ref, h_ref, w_ref, b_ref, o_ref):
    agg = a0_ref[...] + a1_ref[...]
    o_ref[...] = (jnp.dot(agg, w_ref[...],
                          preferred_element_type=jnp.float32)
                  + b_ref[...] + h_ref[...])


def _core_update(agg2, h, w, b):
    n, f = h.shape
    nb = n // _BLK
    return pl.pallas_call(
        _core_body,
        grid=(nb,),
        in_specs=[
            pl.BlockSpec((_BLK, f), lambda i: (i, 0)),
            pl.BlockSpec((_BLK, f), lambda i: (i + nb, 0)),
            pl.BlockSpec((_BLK, f), lambda i: (i, 0)),
            pl.BlockSpec((f, f), lambda i: (0, 0)),
            pl.BlockSpec((1, f), lambda i: (0, 0)),
        ],
        out_specs=pl.BlockSpec((_BLK, f), lambda i: (i, 0)),
        out_shape=jax.ShapeDtypeStruct((n, f), jnp.float32),
    )(agg2, agg2, h, w, b.reshape(1, -1))


def _final_body(a0_ref, a1_ref, h_ref, wc_ref, bc_ref, wd0_ref, bd0_ref,
                wd1_ref, bd1_ref, o_ref):
    hh = (jnp.dot(a0_ref[...] + a1_ref[...], wc_ref[...],
                  preferred_element_type=jnp.float32)
          + bc_ref[...] + h_ref[...])
    hh = jnp.dot(hh, wd0_ref[...],
                 preferred_element_type=jnp.float32) + bd0_ref[...]
    o_ref[...] = jnp.dot(hh, wd1_ref[...],
                         preferred_element_type=jnp.float32) + bd1_ref[...]


def _final(agg2, h, wc, bc, wd0, bd0, wd1p, bd1p):
    n, f = h.shape
    nb = n // _BLK
    return pl.pallas_call(
        _final_body,
        grid=(nb,),
        in_specs=[
            pl.BlockSpec((_BLK, f), lambda i: (i, 0)),
            pl.BlockSpec((_BLK, f), lambda i: (i + nb, 0)),
            pl.BlockSpec((_BLK, f), lambda i: (i, 0)),
            pl.BlockSpec((f, f), lambda i: (0, 0)),
            pl.BlockSpec((1, f), lambda i: (0, 0)),
            pl.BlockSpec((f, f), lambda i: (0, 0)),
            pl.BlockSpec((1, f), lambda i: (0, 0)),
            pl.BlockSpec((f, f), lambda i: (0, 0)),
            pl.BlockSpec((1, f), lambda i: (0, 0)),
        ],
        out_specs=pl.BlockSpec((_BLK, f), lambda i: (i, 0)),
        out_shape=jax.ShapeDtypeStruct((n, f), jnp.float32),
    )(agg2, agg2, h, wc, bc.reshape(1, -1), wd0, bd0.reshape(1, -1),
      wd1p, bd1p.reshape(1, -1))


def kernel(x, edge_index, edge_weight, W_enc0, b_enc0, W_enc1, b_enc1,
           W_core0, b_core0, W_core1, b_core1, W_dec0, b_dec0, W_dec1,
           b_dec1):
    n, f = x.shape
    e = edge_weight.shape[0]
    src = edge_index[1]
    dst = edge_index[0]

    h = _encoder(x, W_enc0, b_enc0, W_enc1, b_enc1)

    agg_fn = _make_agg_kernel(n, f, e)
    agg2 = agg_fn(h, src, dst, edge_weight)
    h = _core_update(agg2, h, W_core0, b_core0)
    agg2 = agg_fn(h, src, dst, edge_weight)

    c = W_dec1.shape[1]
    wd1p = jnp.pad(W_dec1, ((0, 0), (0, f - c)))
    bd1p = jnp.pad(b_dec1, (0, f - c))
    out = _final(agg2, h, W_core1, b_core1, W_dec0, b_dec0, wd1p, bd1p)
    return out[:, :c]


# speedup vs baseline: 11.4385x; 1.0939x over previous
import functools

import jax
import jax.numpy as jnp
from jax import lax
from jax.experimental import pallas as pl
from jax.experimental.pallas import tpu as pltpu
from jax.experimental.pallas import tpu_sc as plsc

_NC, _NS = 2, 16
_NW = _NC * _NS
_CHUNK = 64
_LANES = 16
_RING = 4


def _make_agg_kernel(n, f, e):
    epw = e // _NW
    full = epw // _CHUNK
    rem = epw - full * _CHUNK
    io_tiles = _NS
    while io_tiles > 1 and (n % io_tiles or (n // io_tiles) % 8):
        io_tiles -= 1
    rows_per_tile = n // io_tiles
    rchunk = 40
    while rows_per_tile % rchunk or rchunk % 8:
        rchunk -= 8
    nrc = rows_per_tile // rchunk
    nsub = f // _LANES

    ngrp = full // _RING
    assert ngrp * _RING == full

    mesh = plsc.VectorSubcoreMesh(core_axis_name="c", subcore_axis_name="s")

    scratch = [
        pltpu.VMEM_SHARED((n, f), jnp.float32),
        pltpu.VMEM((rchunk, f), jnp.float32),
        pltpu.VMEM((rchunk, f), jnp.float32),
        pltpu.SemaphoreType.DMA,
    ]
    scratch += [pltpu.VMEM((_CHUNK, f), jnp.float32) for _ in range(_RING)]
    scratch += [pltpu.VMEM((_CHUNK,), jnp.int32) for _ in range(_RING)]
    scratch += [pltpu.VMEM((_CHUNK,), jnp.int32) for _ in range(_RING)]
    scratch += [pltpu.VMEM((_CHUNK,), jnp.float32) for _ in range(_RING)]
    scratch += [
        pltpu.SemaphoreType.DMA((_RING,)),
        pltpu.SemaphoreType.DMA((_RING,)),
        pltpu.SemaphoreType.DMA((_RING,)),
        pltpu.SemaphoreType.DMA((_RING,)),
    ]
    if rem:
        scratch += [
            pltpu.VMEM((rem,), jnp.int32),
            pltpu.VMEM((rem,), jnp.int32),
            pltpu.VMEM((rem,), jnp.float32),
            pltpu.VMEM((rem, f), jnp.float32),
        ]

    @functools.partial(
        pl.kernel,
        mesh=mesh,
        out_type=jax.ShapeDtypeStruct((2 * n, f), jnp.float32),
        scratch_types=scratch,
    )
    def agg(h_hbm, src_hbm, dst_hbm, ew_hbm, out_hbm, acc, stage_v, stage2_v,
            zsem, *bufs):
        rows_r = bufs[:_RING]
        src_r = bufs[_RING:2 * _RING]
        dst_r = bufs[2 * _RING:3 * _RING]
        ew_r = bufs[3 * _RING:4 * _RING]
        gsem, isem, dsem, ssem = bufs[4 * _RING:4 * _RING + 4]
        rem_bufs = bufs[4 * _RING + 4:]
        c = lax.axis_index("c")
        s = lax.axis_index("s")
        wid = c * _NS + s
        row0 = s * rows_per_tile
        ebase = wid * epw

        def srcew_descs(ci, b):
            base = ebase + ci * _CHUNK
            return (
                pltpu.make_async_copy(src_hbm.at[pl.ds(base, _CHUNK)],
                                      src_r[b], isem.at[b]),
                pltpu.make_async_copy(ew_hbm.at[pl.ds(base, _CHUNK)],
                                      ew_r[b], isem.at[b]),
            )

        def dst_desc(ci, b):
            base = ebase + ci * _CHUNK
            return pltpu.make_async_copy(dst_hbm.at[pl.ds(base, _CHUNK)],
                                         dst_r[b], dsem.at[b])

        def issue_srcew(ci, b):
            for d in srcew_descs(ci, b):
                d.start()

        def wait_srcew(ci, b):
            for d in srcew_descs(ci, b):
                d.wait()

        half = _CHUNK // 2

        def gather_descs(b):
            return (
                pltpu.make_async_copy(
                    h_hbm.at[src_r[b].at[pl.ds(0, half)]],
                    rows_r[b].at[pl.ds(0, half)], gsem.at[b]),
                pltpu.make_async_copy(
                    h_hbm.at[src_r[b].at[pl.ds(half, half)]],
                    rows_r[b].at[pl.ds(half, half)], gsem.at[b]),
            )

        def start_gather(b):
            for d in gather_descs(b):
                d.start()

        def wait_gather(b):
            for d in gather_descs(b):
                d.wait()

        def scatter_desc(b):
            return pltpu.make_async_copy(rows_r[b], acc.at[dst_r[b]],
                                         ssem.at[b])

        def issue_scatter(b):
            pltpu.async_copy(rows_r[b], acc.at[dst_r[b]], ssem.at[b],
                             add=True)

        def scale(b):
            def scale_body(g, inner):
                wvec = ew_r[b][pl.ds(g * _LANES, _LANES)]
                for l in range(_LANES):
                    w = jnp.full((_LANES,), wvec[l], jnp.float32)
                    row = g * _LANES + l
                    for k in range(nsub):
                        sl = pl.ds(k * _LANES, _LANES)
                        rows_r[b][row, sl] = rows_r[b][row, sl] * w
                return inner
            lax.fori_loop(0, _CHUNK // _LANES, scale_body, 0)

        issue_srcew(0, 0)
        issue_srcew(1, 1)
        issue_srcew(2, 2)
        issue_srcew(3, 3)
        dst_desc(0, 0).start()
        dst_desc(1, 1).start()

        @pl.when(s < io_tiles)
        def _zero():
            def zero_body(j, carry):
                for k in range(nsub):
                    stage_v[j, pl.ds(k * _LANES, _LANES)] = jnp.zeros(
                        (_LANES,), jnp.float32)
                return carry
            lax.fori_loop(0, rchunk, zero_body, 0)
            for i in range(nrc):
                pltpu.async_copy(stage_v,
                                 acc.at[pl.ds(row0 + i * rchunk, rchunk)],
                                 zsem)
            for i in range(nrc):
                pltpu.make_async_copy(
                    stage_v, acc.at[pl.ds(row0 + i * rchunk, rchunk)],
                    zsem).wait()
        plsc.subcore_barrier()

        wait_srcew(0, 0)
        start_gather(0)
        wait_srcew(1, 1)
        start_gather(1)
        wait_srcew(2, 2)
        start_gather(2)

        def group_body(g9, carry):
            for j in range(_RING):
                b = j
                ci = g9 * _RING + j
                wait_gather(b)
                scale(b)
                pb = (j + 3) % _RING
                if j >= 1:
                    scatter_desc(pb).wait()
                else:
                    @pl.when(g9 > 0)
                    def _ws():
                        scatter_desc(pb).wait()
                if j < 1:
                    wait_srcew(ci + 3, pb)
                    start_gather(pb)
                else:
                    @pl.when(g9 < ngrp - 1)
                    def _wg():
                        wait_srcew(ci + 3, pb)
                        start_gather(pb)
                dst_desc(ci, b).wait()
                issue_scatter(b)
                sb = (j + 2) % _RING
                @pl.when(g9 < ngrp - 1)
                def _wi():
                    issue_srcew(ci + 4, b)
                if j < 2:
                    dst_desc(ci + 2, sb).start()
                else:
                    @pl.when(g9 < ngrp - 1)
                    def _wd():
                        dst_desc(ci + 2, sb).start()
            return carry
        lax.fori_loop(0, ngrp, group_body, 0)
        scatter_desc(_RING - 1).wait()

        if rem:
            srcr_v, dstr_v, ewr_v, rowsr_v = rem_bufs
            base = ebase + full * _CHUNK
            pltpu.sync_copy(src_hbm.at[pl.ds(base, rem)], srcr_v)
            pltpu.sync_copy(dst_hbm.at[pl.ds(base, rem)], dstr_v)
            pltpu.sync_copy(ew_hbm.at[pl.ds(base, rem)], ewr_v)
            pltpu.async_copy(h_hbm.at[srcr_v], rowsr_v, gsem.at[0]).wait()
            for g in range(rem // _LANES):
                wvec = ewr_v[pl.ds(g * _LANES, _LANES)]
                for l in range(_LANES):
                    w = jnp.full((_LANES,), wvec[l], jnp.float32)
                    row = g * _LANES + l
                    for k in range(nsub):
                        sl = pl.ds(k * _LANES, _LANES)
                        rowsr_v[row, sl] = rowsr_v[row, sl] * w
            pltpu.sync_copy(rowsr_v, acc.at[dstr_v], add=True)

        plsc.subcore_barrier()

        @pl.when(s < io_tiles)
        def _writeout():
            out0 = c * n
            stages = (stage_v, stage2_v)
            for i in range(nrc):
                r = row0 + i * rchunk
                bb = stages[i % 2]
                if i >= 2:
                    rp = row0 + (i - 2) * rchunk
                    pltpu.make_async_copy(
                        bb, out_hbm.at[pl.ds(out0 + rp, rchunk)], zsem).wait()
                pltpu.sync_copy(acc.at[pl.ds(r, rchunk)], bb)
                pltpu.async_copy(bb, out_hbm.at[pl.ds(out0 + r, rchunk)], zsem)
            for i in range(max(0, nrc - 2), nrc):
                r = row0 + i * rchunk
                pltpu.make_async_copy(
                    stages[i % 2], out_hbm.at[pl.ds(out0 + r, rchunk)],
                    zsem).wait()

    return agg


_BLK = 2000


def _enc_body(x_ref, w0_ref, b0_ref, w1_ref, b1_ref, o_ref):
    t = jnp.dot(x_ref[...], w0_ref[...],
                preferred_element_type=jnp.float32) + b0_ref[...]
    o_ref[...] = jnp.dot(t, w1_ref[...],
                         preferred_element_type=jnp.float32) + b1_ref[...]


def _encoder(x, w0, b0, w1, b1):
    n, fin = x.shape
    l0, l1 = w0.shape[1], w1.shape[1]
    return pl.pallas_call(
        _enc_body,
        grid=(n // _BLK,),
        in_specs=[
            pl.BlockSpec((_BLK, fin), lambda i: (i, 0)),
            pl.BlockSpec((fin, l0), lambda i: (0, 0)),
            pl.BlockSpec((1, l0), lambda i: (0, 0)),
            pl.BlockSpec((l0, l1), lambda i: (0, 0)),
            pl.BlockSpec((1, l1), lambda i: (0, 0)),
        ],
        out_specs=pl.BlockSpec((_BLK, l1), lambda i: (i, 0)),
        out_shape=jax.ShapeDtypeStruct((n, l1), jnp.float32),
    )(x, w0, b0.reshape(1, -1), w1, b1.reshape(1, -1))


def _core_body(a0_ref, a1_ref, h_ref, w_ref, b_ref, o_ref):
    agg = a0_ref[...] + a1_ref[...]
    o_ref[...] = (jnp.dot(agg, w_ref[...],
                          preferred_element_type=jnp.float32)
                  + b_ref[...] + h_ref[...])


def _core_update(agg2, h, w, b):
    n, f = h.shape
    nb = n // _BLK
    return pl.pallas_call(
        _core_body,
        grid=(nb,),
        in_specs=[
            pl.BlockSpec((_BLK, f), lambda i: (i, 0)),
            pl.BlockSpec((_BLK, f), lambda i: (i + nb, 0)),
            pl.BlockSpec((_BLK, f), lambda i: (i, 0)),
            pl.BlockSpec((f, f), lambda i: (0, 0)),
            pl.BlockSpec((1, f), lambda i: (0, 0)),
        ],
        out_specs=pl.BlockSpec((_BLK, f), lambda i: (i, 0)),
        out_shape=jax.ShapeDtypeStruct((n, f), jnp.float32),
    )(agg2, agg2, h, w, b.reshape(1, -1))


def _final_body(a0_ref, a1_ref, h_ref, wc_ref, bc_ref, wd0_ref, bd0_ref,
                wd1_ref, bd1_ref, o_ref):
    hh = (jnp.dot(a0_ref[...] + a1_ref[...], wc_ref[...],
                  preferred_element_type=jnp.float32)
          + bc_ref[...] + h_ref[...])
    hh = jnp.dot(hh, wd0_ref[...],
                 preferred_element_type=jnp.float32) + bd0_ref[...]
    o_ref[...] = jnp.dot(hh, wd1_ref[...],
                         preferred_element_type=jnp.float32) + bd1_ref[...]


def _final(agg2, h, wc, bc, wd0, bd0, wd1p, bd1p):
    n, f = h.shape
    nb = n // _BLK
    return pl.pallas_call(
        _final_body,
        grid=(nb,),
        in_specs=[
            pl.BlockSpec((_BLK, f), lambda i: (i, 0)),
            pl.BlockSpec((_BLK, f), lambda i: (i + nb, 0)),
            pl.BlockSpec((_BLK, f), lambda i: (i, 0)),
            pl.BlockSpec((f, f), lambda i: (0, 0)),
            pl.BlockSpec((1, f), lambda i: (0, 0)),
            pl.BlockSpec((f, f), lambda i: (0, 0)),
            pl.BlockSpec((1, f), lambda i: (0, 0)),
            pl.BlockSpec((f, f), lambda i: (0, 0)),
            pl.BlockSpec((1, f), lambda i: (0, 0)),
        ],
        out_specs=pl.BlockSpec((_BLK, f), lambda i: (i, 0)),
        out_shape=jax.ShapeDtypeStruct((n, f), jnp.float32),
    )(agg2, agg2, h, wc, bc.reshape(1, -1), wd0, bd0.reshape(1, -1),
      wd1p, bd1p.reshape(1, -1))


def kernel(x, edge_index, edge_weight, W_enc0, b_enc0, W_enc1, b_enc1,
           W_core0, b_core0, W_core1, b_core1, W_dec0, b_dec0, W_dec1,
           b_dec1):
    n, f = x.shape
    e = edge_weight.shape[0]
    src = edge_index[1]
    dst = edge_index[0]

    h = _encoder(x, W_enc0, b_enc0, W_enc1, b_enc1)

    agg_fn = _make_agg_kernel(n, f, e)
    agg2 = agg_fn(h, src, dst, edge_weight)
    h = _core_update(agg2, h, W_core0, b_core0)
    agg2 = agg_fn(h, src, dst, edge_weight)

    c = W_dec1.shape[1]
    wd1p = jnp.pad(W_dec1, ((0, 0), (0, f - c)))
    bd1p = jnp.pad(b_dec1, (0, f - c))
    out = _final(agg2, h, W_core1, b_core1, W_dec0, b_dec0, wd1p, bd1p)
    return out[:, :c]


# speedup vs baseline: 11.4405x; 1.0002x over previous
import functools

import jax
import jax.numpy as jnp
from jax import lax
from jax.experimental import pallas as pl
from jax.experimental.pallas import tpu as pltpu
from jax.experimental.pallas import tpu_sc as plsc

_NC, _NS = 2, 16
_NW = _NC * _NS
_CHUNK = 64
_LANES = 16
_RING = 4


def _make_agg_kernel(n, f, e):
    epw = e // _NW
    full = epw // _CHUNK
    rem = epw - full * _CHUNK
    io_tiles = _NS
    while io_tiles > 1 and (n % io_tiles or (n // io_tiles) % 8):
        io_tiles -= 1
    rows_per_tile = n // io_tiles
    rchunk = 40
    while rows_per_tile % rchunk or rchunk % 8:
        rchunk -= 8
    nrc = rows_per_tile // rchunk
    nsub = f // _LANES

    ngrp = full // _RING
    assert ngrp * _RING == full

    mesh = plsc.VectorSubcoreMesh(core_axis_name="c", subcore_axis_name="s")

    scratch = [
        pltpu.VMEM_SHARED((n, f), jnp.float32),
        pltpu.VMEM((rchunk, f), jnp.float32),
        pltpu.VMEM((rchunk, f), jnp.float32),
        pltpu.SemaphoreType.DMA,
    ]
    scratch += [pltpu.VMEM((_CHUNK, f), jnp.float32) for _ in range(_RING)]
    scratch += [pltpu.VMEM((_CHUNK,), jnp.int32) for _ in range(_RING)]
    scratch += [pltpu.VMEM((_CHUNK,), jnp.int32) for _ in range(_RING)]
    scratch += [pltpu.VMEM((_CHUNK,), jnp.float32) for _ in range(_RING)]
    scratch += [
        pltpu.SemaphoreType.DMA((_RING,)),
        pltpu.SemaphoreType.DMA((_RING,)),
        pltpu.SemaphoreType.DMA((_RING,)),
        pltpu.SemaphoreType.DMA((_RING,)),
    ]
    if rem:
        scratch += [
            pltpu.VMEM((rem,), jnp.int32),
            pltpu.VMEM((rem,), jnp.int32),
            pltpu.VMEM((rem,), jnp.float32),
            pltpu.VMEM((rem, f), jnp.float32),
        ]

    @functools.partial(
        pl.kernel,
        mesh=mesh,
        out_type=jax.ShapeDtypeStruct((2 * n, f), jnp.float32),
        scratch_types=scratch,
    )
    def agg(h_hbm, src_hbm, dst_hbm, ew_hbm, out_hbm, acc, stage_v, stage2_v,
            zsem, *bufs):
        rows_r = bufs[:_RING]
        src_r = bufs[_RING:2 * _RING]
        dst_r = bufs[2 * _RING:3 * _RING]
        ew_r = bufs[3 * _RING:4 * _RING]
        gsem, isem, dsem, ssem = bufs[4 * _RING:4 * _RING + 4]
        rem_bufs = bufs[4 * _RING + 4:]
        c = lax.axis_index("c")
        s = lax.axis_index("s")
        wid = c * _NS + s
        row0 = s * rows_per_tile
        ebase = wid * epw

        def srcew_descs(ci, b):
            base = ebase + ci * _CHUNK
            return (
                pltpu.make_async_copy(src_hbm.at[pl.ds(base, _CHUNK)],
                                      src_r[b], isem.at[b]),
                pltpu.make_async_copy(ew_hbm.at[pl.ds(base, _CHUNK)],
                                      ew_r[b], isem.at[b]),
            )

        def dst_desc(ci, b):
            base = ebase + ci * _CHUNK
            return pltpu.make_async_copy(dst_hbm.at[pl.ds(base, _CHUNK)],
                                         dst_r[b], dsem.at[b])

        def issue_srcew(ci, b):
            for d in srcew_descs(ci, b):
                d.start()

        def wait_srcew(ci, b):
            for d in srcew_descs(ci, b):
                d.wait()

        _NSPLIT = 4
        part = _CHUNK // _NSPLIT

        def gather_descs(b):
            return tuple(
                pltpu.make_async_copy(
                    h_hbm.at[src_r[b].at[pl.ds(q * part, part)]],
                    rows_r[b].at[pl.ds(q * part, part)], gsem.at[b])
                for q in range(_NSPLIT))

        def start_gather(b):
            for d in gather_descs(b):
                d.start()

        def wait_gather(b):
            for d in gather_descs(b):
                d.wait()

        def scatter_desc(b):
            return pltpu.make_async_copy(rows_r[b], acc.at[dst_r[b]],
                                         ssem.at[b])

        def issue_scatter(b):
            pltpu.async_copy(rows_r[b], acc.at[dst_r[b]], ssem.at[b],
                             add=True)

        def scale(b):
            def scale_body(g, inner):
                wvec = ew_r[b][pl.ds(g * _LANES, _LANES)]
                for l in range(_LANES):
                    w = jnp.full((_LANES,), wvec[l], jnp.float32)
                    row = g * _LANES + l
                    for k in range(nsub):
                        sl = pl.ds(k * _LANES, _LANES)
                        rows_r[b][row, sl] = rows_r[b][row, sl] * w
                return inner
            lax.fori_loop(0, _CHUNK // _LANES, scale_body, 0)

        issue_srcew(0, 0)
        issue_srcew(1, 1)
        issue_srcew(2, 2)
        issue_srcew(3, 3)
        dst_desc(0, 0).start()
        dst_desc(1, 1).start()

        @pl.when(s < io_tiles)
        def _zero():
            def zero_body(j, carry):
                for k in range(nsub):
                    stage_v[j, pl.ds(k * _LANES, _LANES)] = jnp.zeros(
                        (_LANES,), jnp.float32)
                return carry
            lax.fori_loop(0, rchunk, zero_body, 0)
            for i in range(nrc):
                pltpu.async_copy(stage_v,
                                 acc.at[pl.ds(row0 + i * rchunk, rchunk)],
                                 zsem)
            for i in range(nrc):
                pltpu.make_async_copy(
                    stage_v, acc.at[pl.ds(row0 + i * rchunk, rchunk)],
                    zsem).wait()
        plsc.subcore_barrier()

        wait_srcew(0, 0)
        start_gather(0)
        wait_srcew(1, 1)
        start_gather(1)
        wait_srcew(2, 2)
        start_gather(2)

        def group_body(g9, carry):
            for j in range(_RING):
                b = j
                ci = g9 * _RING + j
                wait_gather(b)
                scale(b)
                pb = (j + 3) % _RING
                if j >= 1:
                    scatter_desc(pb).wait()
                else:
                    @pl.when(g9 > 0)
                    def _ws():
                        scatter_desc(pb).wait()
                if j < 1:
                    wait_srcew(ci + 3, pb)
                    start_gather(pb)
                else:
                    @pl.when(g9 < ngrp - 1)
                    def _wg():
                        wait_srcew(ci + 3, pb)
                        start_gather(pb)
                dst_desc(ci, b).wait()
                issue_scatter(b)
                sb = (j + 2) % _RING
                @pl.when(g9 < ngrp - 1)
                def _wi():
                    issue_srcew(ci + 4, b)
                if j < 2:
                    dst_desc(ci + 2, sb).start()
                else:
                    @pl.when(g9 < ngrp - 1)
                    def _wd():
                        dst_desc(ci + 2, sb).start()
            return carry
        lax.fori_loop(0, ngrp, group_body, 0)
        scatter_desc(_RING - 1).wait()

        if rem:
            srcr_v, dstr_v, ewr_v, rowsr_v = rem_bufs
            base = ebase + full * _CHUNK
            pltpu.sync_copy(src_hbm.at[pl.ds(base, rem)], srcr_v)
            pltpu.sync_copy(dst_hbm.at[pl.ds(base, rem)], dstr_v)
            pltpu.sync_copy(ew_hbm.at[pl.ds(base, rem)], ewr_v)
            pltpu.async_copy(h_hbm.at[srcr_v], rowsr_v, gsem.at[0]).wait()
            for g in range(rem // _LANES):
                wvec = ewr_v[pl.ds(g * _LANES, _LANES)]
                for l in range(_LANES):
                    w = jnp.full((_LANES,), wvec[l], jnp.float32)
                    row = g * _LANES + l
                    for k in range(nsub):
                        sl = pl.ds(k * _LANES, _LANES)
                        rowsr_v[row, sl] = rowsr_v[row, sl] * w
            pltpu.sync_copy(rowsr_v, acc.at[dstr_v], add=True)

        plsc.subcore_barrier()

        @pl.when(s < io_tiles)
        def _writeout():
            out0 = c * n
            stages = (stage_v, stage2_v)
            for i in range(nrc):
                r = row0 + i * rchunk
                bb = stages[i % 2]
                if i >= 2:
                    rp = row0 + (i - 2) * rchunk
                    pltpu.make_async_copy(
                        bb, out_hbm.at[pl.ds(out0 + rp, rchunk)], zsem).wait()
                pltpu.sync_copy(acc.at[pl.ds(r, rchunk)], bb)
                pltpu.async_copy(bb, out_hbm.at[pl.ds(out0 + r, rchunk)], zsem)
            for i in range(max(0, nrc - 2), nrc):
                r = row0 + i * rchunk
                pltpu.make_async_copy(
                    stages[i % 2], out_hbm.at[pl.ds(out0 + r, rchunk)],
                    zsem).wait()

    return agg


_BLK = 2000


def _enc_body(x_ref, w0_ref, b0_ref, w1_ref, b1_ref, o_ref):
    t = jnp.dot(x_ref[...], w0_ref[...],
                preferred_element_type=jnp.float32) + b0_ref[...]
    o_ref[...] = jnp.dot(t, w1_ref[...],
                         preferred_element_type=jnp.float32) + b1_ref[...]


def _encoder(x, w0, b0, w1, b1):
    n, fin = x.shape
    l0, l1 = w0.shape[1], w1.shape[1]
    return pl.pallas_call(
        _enc_body,
        grid=(n // _BLK,),
        in_specs=[
            pl.BlockSpec((_BLK, fin), lambda i: (i, 0)),
            pl.BlockSpec((fin, l0), lambda i: (0, 0)),
            pl.BlockSpec((1, l0), lambda i: (0, 0)),
            pl.BlockSpec((l0, l1), lambda i: (0, 0)),
            pl.BlockSpec((1, l1), lambda i: (0, 0)),
        ],
        out_specs=pl.BlockSpec((_BLK, l1), lambda i: (i, 0)),
        out_shape=jax.ShapeDtypeStruct((n, l1), jnp.float32),
    )(x, w0, b0.reshape(1, -1), w1, b1.reshape(1, -1))


def _core_body(a0_ref, a1_ref, h_ref, w_ref, b_ref, o_ref):
    agg = a0_ref[...] + a1_ref[...]
    o_ref[...] = (jnp.dot(agg, w_ref[...],
                          preferred_element_type=jnp.float32)
                  + b_ref[...] + h_ref[...])


def _core_update(agg2, h, w, b):
    n, f = h.shape
    nb = n // _BLK
    return pl.pallas_call(
        _core_body,
        grid=(nb,),
        in_specs=[
            pl.BlockSpec((_BLK, f), lambda i: (i, 0)),
            pl.BlockSpec((_BLK, f), lambda i: (i + nb, 0)),
            pl.BlockSpec((_BLK, f), lambda i: (i, 0)),
            pl.BlockSpec((f, f), lambda i: (0, 0)),
            pl.BlockSpec((1, f), lambda i: (0, 0)),
        ],
        out_specs=pl.BlockSpec((_BLK, f), lambda i: (i, 0)),
        out_shape=jax.ShapeDtypeStruct((n, f), jnp.float32),
    )(agg2, agg2, h, w, b.reshape(1, -1))


def _final_body(a0_ref, a1_ref, h_ref, wc_ref, bc_ref, wd0_ref, bd0_ref,
                wd1_ref, bd1_ref, o_ref):
    hh = (jnp.dot(a0_ref[...] + a1_ref[...], wc_ref[...],
                  preferred_element_type=jnp.float32)
          + bc_ref[...] + h_ref[...])
    hh = jnp.dot(hh, wd0_ref[...],
                 preferred_element_type=jnp.float32) + bd0_ref[...]
    o_ref[...] = jnp.dot(hh, wd1_ref[...],
                         preferred_element_type=jnp.float32) + bd1_ref[...]


def _final(agg2, h, wc, bc, wd0, bd0, wd1p, bd1p):
    n, f = h.shape
    nb = n // _BLK
    return pl.pallas_call(
        _final_body,
        grid=(nb,),
        in_specs=[
            pl.BlockSpec((_BLK, f), lambda i: (i, 0)),
            pl.BlockSpec((_BLK, f), lambda i: (i + nb, 0)),
            pl.BlockSpec((_BLK, f), lambda i: (i, 0)),
            pl.BlockSpec((f, f), lambda i: (0, 0)),
            pl.BlockSpec((1, f), lambda i: (0, 0)),
            pl.BlockSpec((f, f), lambda i: (0, 0)),
            pl.BlockSpec((1, f), lambda i: (0, 0)),
            pl.BlockSpec((f, f), lambda i: (0, 0)),
            pl.BlockSpec((1, f), lambda i: (0, 0)),
        ],
        out_specs=pl.BlockSpec((_BLK, f), lambda i: (i, 0)),
        out_shape=jax.ShapeDtypeStruct((n, f), jnp.float32),
    )(agg2, agg2, h, wc, bc.reshape(1, -1), wd0, bd0.reshape(1, -1),
      wd1p, bd1p.reshape(1, -1))


def kernel(x, edge_index, edge_weight, W_enc0, b_enc0, W_enc1, b_enc1,
           W_core0, b_core0, W_core1, b_core1, W_dec0, b_dec0, W_dec1,
           b_dec1):
    n, f = x.shape
    e = edge_weight.shape[0]
    src = edge_index[1]
    dst = edge_index[0]

    h = _encoder(x, W_enc0, b_enc0, W_enc1, b_enc1)

    agg_fn = _make_agg_kernel(n, f, e)
    agg2 = agg_fn(h, src, dst, edge_weight)
    h = _core_update(agg2, h, W_core0, b_core0)
    agg2 = agg_fn(h, src, dst, edge_weight)

    c = W_dec1.shape[1]
    wd1p = jnp.pad(W_dec1, ((0, 0), (0, f - c)))
    bd1p = jnp.pad(b_dec1, (0, f - c))
    out = _final(agg2, h, W_core1, b_core1, W_dec0, b_dec0, wd1p, bd1p)
    return out[:, :c]


# speedup vs baseline: 11.4770x; 1.0032x over previous
import functools

import jax
import jax.numpy as jnp
from jax import lax
from jax.experimental import pallas as pl
from jax.experimental.pallas import tpu as pltpu
from jax.experimental.pallas import tpu_sc as plsc

_NC, _NS = 2, 16
_NW = _NC * _NS
_CHUNK = 64
_LANES = 16
_RING = 4


def _make_agg_kernel(n, f, e):
    epw = e // _NW
    full = epw // _CHUNK
    rem = epw - full * _CHUNK
    io_tiles = _NS
    while io_tiles > 1 and (n % io_tiles or (n // io_tiles) % 8):
        io_tiles -= 1
    rows_per_tile = n // io_tiles
    rchunk = 40
    while rows_per_tile % rchunk or rchunk % 8:
        rchunk -= 8
    nrc = rows_per_tile // rchunk
    nsub = f // _LANES

    ngrp = full // _RING
    assert ngrp * _RING == full

    mesh = plsc.VectorSubcoreMesh(core_axis_name="c", subcore_axis_name="s")

    scratch = [
        pltpu.VMEM_SHARED((n, f), jnp.float32),
        pltpu.VMEM((rchunk, f), jnp.float32),
        pltpu.VMEM((rchunk, f), jnp.float32),
        pltpu.SemaphoreType.DMA,
    ]
    scratch += [pltpu.VMEM((_CHUNK, f), jnp.float32) for _ in range(_RING)]
    scratch += [pltpu.VMEM((_CHUNK,), jnp.int32) for _ in range(_RING)]
    scratch += [pltpu.VMEM((_CHUNK,), jnp.int32) for _ in range(_RING)]
    scratch += [pltpu.VMEM((_CHUNK,), jnp.float32) for _ in range(_RING)]
    scratch += [
        pltpu.SemaphoreType.DMA((_RING,)),
        pltpu.SemaphoreType.DMA((_RING,)),
        pltpu.SemaphoreType.DMA((_RING,)),
        pltpu.SemaphoreType.DMA((_RING,)),
    ]
    if rem:
        scratch += [
            pltpu.VMEM((rem,), jnp.int32),
            pltpu.VMEM((rem,), jnp.int32),
            pltpu.VMEM((rem,), jnp.float32),
            pltpu.VMEM((rem, f), jnp.float32),
        ]

    @functools.partial(
        pl.kernel,
        mesh=mesh,
        out_type=jax.ShapeDtypeStruct((2 * n, f), jnp.float32),
        scratch_types=scratch,
    )
    def agg(h_hbm, src_hbm, dst_hbm, ew_hbm, out_hbm, acc, stage_v, stage2_v,
            zsem, *bufs):
        rows_r = bufs[:_RING]
        src_r = bufs[_RING:2 * _RING]
        dst_r = bufs[2 * _RING:3 * _RING]
        ew_r = bufs[3 * _RING:4 * _RING]
        gsem, isem, dsem, ssem = bufs[4 * _RING:4 * _RING + 4]
        rem_bufs = bufs[4 * _RING + 4:]
        c = lax.axis_index("c")
        s = lax.axis_index("s")
        wid = c * _NS + s
        row0 = s * rows_per_tile
        ebase = wid * epw

        def srcew_descs(ci, b):
            base = ebase + ci * _CHUNK
            return (
                pltpu.make_async_copy(src_hbm.at[pl.ds(base, _CHUNK)],
                                      src_r[b], isem.at[b]),
                pltpu.make_async_copy(ew_hbm.at[pl.ds(base, _CHUNK)],
                                      ew_r[b], isem.at[b]),
            )

        def dst_desc(ci, b):
            base = ebase + ci * _CHUNK
            return pltpu.make_async_copy(dst_hbm.at[pl.ds(base, _CHUNK)],
                                         dst_r[b], dsem.at[b])

        def issue_srcew(ci, b):
            for d in srcew_descs(ci, b):
                d.start()

        def wait_srcew(ci, b):
            for d in srcew_descs(ci, b):
                d.wait()

        _NSPLIT = 4
        part = _CHUNK // _NSPLIT

        def gather_descs(b):
            return tuple(
                pltpu.make_async_copy(
                    h_hbm.at[src_r[b].at[pl.ds(q * part, part)]],
                    rows_r[b].at[pl.ds(q * part, part)], gsem.at[b])
                for q in range(_NSPLIT))

        def start_gather(b):
            for d in gather_descs(b):
                d.start()

        def wait_gather(b):
            for d in gather_descs(b):
                d.wait()

        def scatter_desc(b):
            return pltpu.make_async_copy(rows_r[b], acc.at[dst_r[b]],
                                         ssem.at[b])

        def issue_scatter(b):
            pltpu.async_copy(rows_r[b], acc.at[dst_r[b]], ssem.at[b],
                             add=True)

        def scale(b):
            def scale_body(g, inner):
                wvec = ew_r[b][pl.ds(g * _LANES, _LANES)]
                for l in range(_LANES):
                    w = jnp.full((_LANES,), wvec[l], jnp.float32)
                    row = g * _LANES + l
                    for k in range(nsub):
                        sl = pl.ds(k * _LANES, _LANES)
                        rows_r[b][row, sl] = rows_r[b][row, sl] * w
                return inner
            lax.fori_loop(0, _CHUNK // _LANES, scale_body, 0)

        issue_srcew(0, 0)
        issue_srcew(1, 1)
        issue_srcew(2, 2)
        issue_srcew(3, 3)
        dst_desc(0, 0).start()
        dst_desc(1, 1).start()

        @pl.when(s < io_tiles)
        def _zero():
            def zero_body(j, carry):
                for k in range(nsub):
                    stage_v[j, pl.ds(k * _LANES, _LANES)] = jnp.zeros(
                        (_LANES,), jnp.float32)
                return carry
            lax.fori_loop(0, rchunk, zero_body, 0)
            for i in range(nrc):
                pltpu.async_copy(stage_v,
                                 acc.at[pl.ds(row0 + i * rchunk, rchunk)],
                                 zsem)
            for i in range(nrc):
                pltpu.make_async_copy(
                    stage_v, acc.at[pl.ds(row0 + i * rchunk, rchunk)],
                    zsem).wait()
        wait_srcew(0, 0)
        start_gather(0)
        wait_srcew(1, 1)
        start_gather(1)
        wait_srcew(2, 2)
        start_gather(2)
        plsc.subcore_barrier()

        def group_body(g9, carry):
            for j in range(_RING):
                b = j
                ci = g9 * _RING + j
                wait_gather(b)
                scale(b)
                pb = (j + 3) % _RING
                if j >= 1:
                    scatter_desc(pb).wait()
                else:
                    @pl.when(g9 > 0)
                    def _ws():
                        scatter_desc(pb).wait()
                if j < 1:
                    wait_srcew(ci + 3, pb)
                    start_gather(pb)
                else:
                    @pl.when(g9 < ngrp - 1)
                    def _wg():
                        wait_srcew(ci + 3, pb)
                        start_gather(pb)
                dst_desc(ci, b).wait()
                issue_scatter(b)
                sb = (j + 2) % _RING
                @pl.when(g9 < ngrp - 1)
                def _wi():
                    issue_srcew(ci + 4, b)
                if j < 2:
                    dst_desc(ci + 2, sb).start()
                else:
                    @pl.when(g9 < ngrp - 1)
                    def _wd():
                        dst_desc(ci + 2, sb).start()
            return carry
        lax.fori_loop(0, ngrp, group_body, 0)
        scatter_desc(_RING - 1).wait()

        if rem:
            srcr_v, dstr_v, ewr_v, rowsr_v = rem_bufs
            base = ebase + full * _CHUNK
            pltpu.sync_copy(src_hbm.at[pl.ds(base, rem)], srcr_v)
            pltpu.sync_copy(dst_hbm.at[pl.ds(base, rem)], dstr_v)
            pltpu.sync_copy(ew_hbm.at[pl.ds(base, rem)], ewr_v)
            pltpu.async_copy(h_hbm.at[srcr_v], rowsr_v, gsem.at[0]).wait()
            for g in range(rem // _LANES):
                wvec = ewr_v[pl.ds(g * _LANES, _LANES)]
                for l in range(_LANES):
                    w = jnp.full((_LANES,), wvec[l], jnp.float32)
                    row = g * _LANES + l
                    for k in range(nsub):
                        sl = pl.ds(k * _LANES, _LANES)
                        rowsr_v[row, sl] = rowsr_v[row, sl] * w
            pltpu.sync_copy(rowsr_v, acc.at[dstr_v], add=True)

        plsc.subcore_barrier()

        @pl.when(s < io_tiles)
        def _writeout():
            out0 = c * n
            stages = (stage_v, stage2_v)
            for i in range(nrc):
                r = row0 + i * rchunk
                bb = stages[i % 2]
                if i >= 2:
                    rp = row0 + (i - 2) * rchunk
                    pltpu.make_async_copy(
                        bb, out_hbm.at[pl.ds(out0 + rp, rchunk)], zsem).wait()
                pltpu.sync_copy(acc.at[pl.ds(r, rchunk)], bb)
                pltpu.async_copy(bb, out_hbm.at[pl.ds(out0 + r, rchunk)], zsem)
            for i in range(max(0, nrc - 2), nrc):
                r = row0 + i * rchunk
                pltpu.make_async_copy(
                    stages[i % 2], out_hbm.at[pl.ds(out0 + r, rchunk)],
                    zsem).wait()

    return agg


_BLK = 2000


def _enc_body(x_ref, w0_ref, b0_ref, w1_ref, b1_ref, o_ref):
    t = jnp.dot(x_ref[...], w0_ref[...],
                preferred_element_type=jnp.float32) + b0_ref[...]
    o_ref[...] = jnp.dot(t, w1_ref[...],
                         preferred_element_type=jnp.float32) + b1_ref[...]


def _encoder(x, w0, b0, w1, b1):
    n, fin = x.shape
    l0, l1 = w0.shape[1], w1.shape[1]
    return pl.pallas_call(
        _enc_body,
        grid=(n // _BLK,),
        in_specs=[
            pl.BlockSpec((_BLK, fin), lambda i: (i, 0)),
            pl.BlockSpec((fin, l0), lambda i: (0, 0)),
            pl.BlockSpec((1, l0), lambda i: (0, 0)),
            pl.BlockSpec((l0, l1), lambda i: (0, 0)),
            pl.BlockSpec((1, l1), lambda i: (0, 0)),
        ],
        out_specs=pl.BlockSpec((_BLK, l1), lambda i: (i, 0)),
        out_shape=jax.ShapeDtypeStruct((n, l1), jnp.float32),
    )(x, w0, b0.reshape(1, -1), w1, b1.reshape(1, -1))


def _core_body(a0_ref, a1_ref, h_ref, w_ref, b_ref, o_ref):
    agg = a0_ref[...] + a1_ref[...]
    o_ref[...] = (jnp.dot(agg, w_ref[...],
                          preferred_element_type=jnp.float32)
                  + b_ref[...] + h_ref[...])


def _core_update(agg2, h, w, b):
    n, f = h.shape
    nb = n // _BLK
    return pl.pallas_call(
        _core_body,
        grid=(nb,),
        in_specs=[
            pl.BlockSpec((_BLK, f), lambda i: (i, 0)),
            pl.BlockSpec((_BLK, f), lambda i: (i + nb, 0)),
            pl.BlockSpec((_BLK, f), lambda i: (i, 0)),
            pl.BlockSpec((f, f), lambda i: (0, 0)),
            pl.BlockSpec((1, f), lambda i: (0, 0)),
        ],
        out_specs=pl.BlockSpec((_BLK, f), lambda i: (i, 0)),
        out_shape=jax.ShapeDtypeStruct((n, f), jnp.float32),
    )(agg2, agg2, h, w, b.reshape(1, -1))


def _final_body(a0_ref, a1_ref, h_ref, wc_ref, bc_ref, wd0_ref, bd0_ref,
                wd1_ref, bd1_ref, o_ref):
    hh = (jnp.dot(a0_ref[...] + a1_ref[...], wc_ref[...],
                  preferred_element_type=jnp.float32)
          + bc_ref[...] + h_ref[...])
    hh = jnp.dot(hh, wd0_ref[...],
                 preferred_element_type=jnp.float32) + bd0_ref[...]
    o_ref[...] = jnp.dot(hh, wd1_ref[...],
                         preferred_element_type=jnp.float32) + bd1_ref[...]


def _final(agg2, h, wc, bc, wd0, bd0, wd1, bd1):
    n, f = h.shape
    cdim = wd1.shape[1]
    nb = n // _BLK
    return pl.pallas_call(
        _final_body,
        grid=(nb,),
        in_specs=[
            pl.BlockSpec((_BLK, f), lambda i: (i, 0)),
            pl.BlockSpec((_BLK, f), lambda i: (i + nb, 0)),
            pl.BlockSpec((_BLK, f), lambda i: (i, 0)),
            pl.BlockSpec((f, f), lambda i: (0, 0)),
            pl.BlockSpec((1, f), lambda i: (0, 0)),
            pl.BlockSpec((f, f), lambda i: (0, 0)),
            pl.BlockSpec((1, f), lambda i: (0, 0)),
            pl.BlockSpec((f, cdim), lambda i: (0, 0)),
            pl.BlockSpec((1, cdim), lambda i: (0, 0)),
        ],
        out_specs=pl.BlockSpec((_BLK, cdim), lambda i: (i, 0)),
        out_shape=jax.ShapeDtypeStruct((n, cdim), jnp.float32),
    )(agg2, agg2, h, wc, bc.reshape(1, -1), wd0, bd0.reshape(1, -1),
      wd1, bd1.reshape(1, -1))


def kernel(x, edge_index, edge_weight, W_enc0, b_enc0, W_enc1, b_enc1,
           W_core0, b_core0, W_core1, b_core1, W_dec0, b_dec0, W_dec1,
           b_dec1):
    n, f = x.shape
    e = edge_weight.shape[0]
    src = edge_index[1]
    dst = edge_index[0]

    h = _encoder(x, W_enc0, b_enc0, W_enc1, b_enc1)

    agg_fn = _make_agg_kernel(n, f, e)
    agg2 = agg_fn(h, src, dst, edge_weight)
    h = _core_update(agg2, h, W_core0, b_core0)
    agg2 = agg_fn(h, src, dst, edge_weight)

    return _final(agg2, h, W_core1, b_core1, W_dec0, b_dec0, W_dec1,
                  b_dec1)


# speedup vs baseline: 11.6036x; 1.0110x over previous
import functools

import jax
import jax.numpy as jnp
from jax import lax
from jax.experimental import pallas as pl
from jax.experimental.pallas import tpu as pltpu
from jax.experimental.pallas import tpu_sc as plsc

_NC, _NS = 2, 16
_NW = _NC * _NS
_CHUNK = 64
_LANES = 16
_RING = 4


def _make_agg_kernel(n, f, e):
    epw = e // _NW
    full = epw // _CHUNK
    rem = epw - full * _CHUNK
    io_tiles = _NS
    while io_tiles > 1 and (n % io_tiles or (n // io_tiles) % 8):
        io_tiles -= 1
    rows_per_tile = n // io_tiles
    rchunk = 40
    while rows_per_tile % rchunk or rchunk % 8:
        rchunk -= 8
    nrc = rows_per_tile // rchunk
    nsub = f // _LANES

    ngrp = full // _RING
    assert ngrp * _RING == full

    mesh = plsc.VectorSubcoreMesh(core_axis_name="c", subcore_axis_name="s")

    scratch = [
        pltpu.VMEM_SHARED((n, f), jnp.float32),
        pltpu.VMEM((rchunk, f), jnp.float32),
        pltpu.VMEM((rchunk, f), jnp.float32),
        pltpu.SemaphoreType.DMA,
    ]
    scratch += [pltpu.VMEM((_CHUNK, f), jnp.float32) for _ in range(_RING)]
    scratch += [pltpu.VMEM((_CHUNK,), jnp.int32) for _ in range(_RING)]
    scratch += [pltpu.VMEM((_CHUNK,), jnp.int32) for _ in range(_RING)]
    scratch += [pltpu.VMEM((_CHUNK,), jnp.float32) for _ in range(_RING)]
    scratch += [
        pltpu.SemaphoreType.DMA((_RING,)),
        pltpu.SemaphoreType.DMA((_RING,)),
        pltpu.SemaphoreType.DMA((_RING,)),
        pltpu.SemaphoreType.DMA((_RING,)),
    ]
    if rem:
        scratch += [
            pltpu.VMEM((rem,), jnp.int32),
            pltpu.VMEM((rem,), jnp.int32),
            pltpu.VMEM((rem,), jnp.float32),
            pltpu.VMEM((rem, f), jnp.float32),
        ]

    @functools.partial(
        pl.kernel,
        mesh=mesh,
        out_type=jax.ShapeDtypeStruct((2 * n, f), jnp.float32),
        scratch_types=scratch,
    )
    def agg(h_hbm, src_hbm, dst_hbm, ew_hbm, out_hbm, acc, stage_v, stage2_v,
            zsem, *bufs):
        rows_r = bufs[:_RING]
        src_r = bufs[_RING:2 * _RING]
        dst_r = bufs[2 * _RING:3 * _RING]
        ew_r = bufs[3 * _RING:4 * _RING]
        gsem, isem, dsem, ssem = bufs[4 * _RING:4 * _RING + 4]
        rem_bufs = bufs[4 * _RING + 4:]
        c = lax.axis_index("c")
        s = lax.axis_index("s")
        wid = c * _NS + s
        row0 = s * rows_per_tile
        ebase = wid * epw

        def srcew_descs(ci, b):
            base = ebase + ci * _CHUNK
            return (
                pltpu.make_async_copy(src_hbm.at[pl.ds(base, _CHUNK)],
                                      src_r[b], isem.at[b]),
                pltpu.make_async_copy(ew_hbm.at[pl.ds(base, _CHUNK)],
                                      ew_r[b], isem.at[b]),
            )

        def dst_desc(ci, b):
            base = ebase + ci * _CHUNK
            return pltpu.make_async_copy(dst_hbm.at[pl.ds(base, _CHUNK)],
                                         dst_r[b], dsem.at[b])

        def issue_srcew(ci, b):
            for d in srcew_descs(ci, b):
                d.start()

        def wait_srcew(ci, b):
            for d in srcew_descs(ci, b):
                d.wait()

        _NSPLIT = 4
        part = _CHUNK // _NSPLIT

        def gather_descs(b):
            return tuple(
                pltpu.make_async_copy(
                    h_hbm.at[src_r[b].at[pl.ds(q * part, part)]],
                    rows_r[b].at[pl.ds(q * part, part)], gsem.at[b])
                for q in range(_NSPLIT))

        def start_gather(b):
            for d in gather_descs(b):
                d.start()

        def wait_gather(b):
            for d in gather_descs(b):
                d.wait()

        def scatter_desc(b):
            return pltpu.make_async_copy(rows_r[b], acc.at[dst_r[b]],
                                         ssem.at[b])

        def issue_scatter(b):
            pltpu.async_copy(rows_r[b], acc.at[dst_r[b]], ssem.at[b],
                             add=True)

        def scale(b):
            def scale_body(g, inner):
                wvec = ew_r[b][pl.ds(g * _LANES, _LANES)]
                for l in range(_LANES):
                    w = jnp.full((_LANES,), wvec[l], jnp.float32)
                    row = g * _LANES + l
                    for k in range(nsub):
                        sl = pl.ds(k * _LANES, _LANES)
                        rows_r[b][row, sl] = rows_r[b][row, sl] * w
                return inner
            lax.fori_loop(0, _CHUNK // _LANES, scale_body, 0)

        issue_srcew(0, 0)
        issue_srcew(1, 1)
        issue_srcew(2, 2)
        issue_srcew(3, 3)
        dst_desc(0, 0).start()
        dst_desc(1, 1).start()

        @pl.when(s < io_tiles)
        def _zero():
            def zero_body(j, carry):
                for k in range(nsub):
                    stage_v[j, pl.ds(k * _LANES, _LANES)] = jnp.zeros(
                        (_LANES,), jnp.float32)
                return carry
            lax.fori_loop(0, rchunk, zero_body, 0)
            for i in range(nrc):
                pltpu.async_copy(stage_v,
                                 acc.at[pl.ds(row0 + i * rchunk, rchunk)],
                                 zsem)
            for i in range(nrc):
                pltpu.make_async_copy(
                    stage_v, acc.at[pl.ds(row0 + i * rchunk, rchunk)],
                    zsem).wait()
        wait_srcew(0, 0)
        start_gather(0)
        wait_srcew(1, 1)
        start_gather(1)
        wait_srcew(2, 2)
        start_gather(2)
        plsc.subcore_barrier()

        def group_body(g9, carry):
            for j in range(_RING):
                b = j
                ci = g9 * _RING + j
                wait_gather(b)
                scale(b)
                pb = (j + 3) % _RING
                if j >= 1:
                    scatter_desc(pb).wait()
                else:
                    @pl.when(g9 > 0)
                    def _ws():
                        scatter_desc(pb).wait()
                if j < 1:
                    wait_srcew(ci + 3, pb)
                    start_gather(pb)
                else:
                    @pl.when(g9 < ngrp - 1)
                    def _wg():
                        wait_srcew(ci + 3, pb)
                        start_gather(pb)
                dst_desc(ci, b).wait()
                issue_scatter(b)
                sb = (j + 2) % _RING
                @pl.when(g9 < ngrp - 1)
                def _wi():
                    issue_srcew(ci + 4, b)
                if j < 2:
                    dst_desc(ci + 2, sb).start()
                else:
                    @pl.when(g9 < ngrp - 1)
                    def _wd():
                        dst_desc(ci + 2, sb).start()
            return carry
        lax.fori_loop(0, ngrp, group_body, 0)
        scatter_desc(_RING - 1).wait()

        if rem:
            srcr_v, dstr_v, ewr_v, rowsr_v = rem_bufs
            base = ebase + full * _CHUNK
            pltpu.sync_copy(src_hbm.at[pl.ds(base, rem)], srcr_v)
            pltpu.sync_copy(dst_hbm.at[pl.ds(base, rem)], dstr_v)
            pltpu.sync_copy(ew_hbm.at[pl.ds(base, rem)], ewr_v)
            pltpu.async_copy(h_hbm.at[srcr_v], rowsr_v, gsem.at[0]).wait()
            for g in range(rem // _LANES):
                wvec = ewr_v[pl.ds(g * _LANES, _LANES)]
                for l in range(_LANES):
                    w = jnp.full((_LANES,), wvec[l], jnp.float32)
                    row = g * _LANES + l
                    for k in range(nsub):
                        sl = pl.ds(k * _LANES, _LANES)
                        rowsr_v[row, sl] = rowsr_v[row, sl] * w
            pltpu.sync_copy(rowsr_v, acc.at[dstr_v], add=True)

        plsc.subcore_barrier()

        @pl.when(s < io_tiles)
        def _writeout():
            out0 = c * n
            pltpu.sync_copy(
                acc.at[pl.ds(row0, rows_per_tile)],
                out_hbm.at[pl.ds(out0 + row0, rows_per_tile)])

    return agg


_BLK = 2000


def _enc_body(x_ref, w0_ref, b0_ref, w1_ref, b1_ref, o_ref):
    t = jnp.dot(x_ref[...], w0_ref[...],
                preferred_element_type=jnp.float32) + b0_ref[...]
    o_ref[...] = jnp.dot(t, w1_ref[...],
                         preferred_element_type=jnp.float32) + b1_ref[...]


def _encoder(x, w0, b0, w1, b1):
    n, fin = x.shape
    l0, l1 = w0.shape[1], w1.shape[1]
    return pl.pallas_call(
        _enc_body,
        grid=(n // _BLK,),
        in_specs=[
            pl.BlockSpec((_BLK, fin), lambda i: (i, 0)),
            pl.BlockSpec((fin, l0), lambda i: (0, 0)),
            pl.BlockSpec((1, l0), lambda i: (0, 0)),
            pl.BlockSpec((l0, l1), lambda i: (0, 0)),
            pl.BlockSpec((1, l1), lambda i: (0, 0)),
        ],
        out_specs=pl.BlockSpec((_BLK, l1), lambda i: (i, 0)),
        out_shape=jax.ShapeDtypeStruct((n, l1), jnp.float32),
    )(x, w0, b0.reshape(1, -1), w1, b1.reshape(1, -1))


def _core_body(a0_ref, a1_ref, h_ref, w_ref, b_ref, o_ref):
    agg = a0_ref[...] + a1_ref[...]
    o_ref[...] = (jnp.dot(agg, w_ref[...],
                          preferred_element_type=jnp.float32)
                  + b_ref[...] + h_ref[...])


def _core_update(agg2, h, w, b):
    n, f = h.shape
    nb = n // _BLK
    return pl.pallas_call(
        _core_body,
        grid=(nb,),
        in_specs=[
            pl.BlockSpec((_BLK, f), lambda i: (i, 0)),
            pl.BlockSpec((_BLK, f), lambda i: (i + nb, 0)),
            pl.BlockSpec((_BLK, f), lambda i: (i, 0)),
            pl.BlockSpec((f, f), lambda i: (0, 0)),
            pl.BlockSpec((1, f), lambda i: (0, 0)),
        ],
        out_specs=pl.BlockSpec((_BLK, f), lambda i: (i, 0)),
        out_shape=jax.ShapeDtypeStruct((n, f), jnp.float32),
    )(agg2, agg2, h, w, b.reshape(1, -1))


def _final_body(a0_ref, a1_ref, h_ref, wc_ref, bc_ref, wd0_ref, bd0_ref,
                wd1_ref, bd1_ref, o_ref):
    hh = (jnp.dot(a0_ref[...] + a1_ref[...], wc_ref[...],
                  preferred_element_type=jnp.float32)
          + bc_ref[...] + h_ref[...])
    hh = jnp.dot(hh, wd0_ref[...],
                 preferred_element_type=jnp.float32) + bd0_ref[...]
    o_ref[...] = jnp.dot(hh, wd1_ref[...],
                         preferred_element_type=jnp.float32) + bd1_ref[...]


def _final(agg2, h, wc, bc, wd0, bd0, wd1, bd1):
    n, f = h.shape
    cdim = wd1.shape[1]
    nb = n // _BLK
    return pl.pallas_call(
        _final_body,
        grid=(nb,),
        in_specs=[
            pl.BlockSpec((_BLK, f), lambda i: (i, 0)),
            pl.BlockSpec((_BLK, f), lambda i: (i + nb, 0)),
            pl.BlockSpec((_BLK, f), lambda i: (i, 0)),
            pl.BlockSpec((f, f), lambda i: (0, 0)),
            pl.BlockSpec((1, f), lambda i: (0, 0)),
            pl.BlockSpec((f, f), lambda i: (0, 0)),
            pl.BlockSpec((1, f), lambda i: (0, 0)),
            pl.BlockSpec((f, cdim), lambda i: (0, 0)),
            pl.BlockSpec((1, cdim), lambda i: (0, 0)),
        ],
        out_specs=pl.BlockSpec((_BLK, cdim), lambda i: (i, 0)),
        out_shape=jax.ShapeDtypeStruct((n, cdim), jnp.float32),
    )(agg2, agg2, h, wc, bc.reshape(1, -1), wd0, bd0.reshape(1, -1),
      wd1, bd1.reshape(1, -1))


def kernel(x, edge_index, edge_weight, W_enc0, b_enc0, W_enc1, b_enc1,
           W_core0, b_core0, W_core1, b_core1, W_dec0, b_dec0, W_dec1,
           b_dec1):
    n, f = x.shape
    e = edge_weight.shape[0]
    src = edge_index[1]
    dst = edge_index[0]

    h = _encoder(x, W_enc0, b_enc0, W_enc1, b_enc1)

    agg_fn = _make_agg_kernel(n, f, e)
    agg2 = agg_fn(h, src, dst, edge_weight)
    h = _core_update(agg2, h, W_core0, b_core0)
    agg2 = agg_fn(h, src, dst, edge_weight)

    return _final(agg2, h, W_core1, b_core1, W_dec0, b_dec0, W_dec1,
                  b_dec1)
